# Initial kernel scaffold; baseline (speedup 1.0000x reference)
import jax
import jax.numpy as jnp
from jax.experimental import pallas as pl


def kernel(edge_index, edge_type, dynamic_emb, emb_rel, W_ih, W_hh, b_ih, b_hh, w_neighbor, loop_weight, time_gate_weight, time_gate_bias):
    raise NotImplementedError("write your pallas kernel here")


# speedup vs baseline: 3.6658x; 3.6658x over previous
import functools

import jax
import jax.numpy as jnp
from jax import lax
from jax.experimental import pallas as pl
from jax.experimental.pallas import tpu as pltpu
from jax.experimental.pallas import tpu_sc as plsc

F32 = jnp.float32

NC = 2
NS = 16
NW = NC * NS
CHUNK = 64


def _l2norm(x):
    n = jnp.sqrt(jnp.sum(x * x, axis=-1, keepdims=True))
    return x / jnp.clip(n, 1e-12, None)


def _tc_norm_body(x_ref, o_ref):
    o_ref[...] = _l2norm(x_ref[...])


def _tc_norm(x, block=1024):
    m, d = x.shape
    grid = m // block
    return pl.pallas_call(
        _tc_norm_body,
        grid=(grid,),
        in_specs=[pl.BlockSpec((block, d), lambda i: (i, 0))],
        out_specs=pl.BlockSpec((block, d), lambda i: (i, 0)),
        out_shape=jax.ShapeDtypeStruct((m, d), F32),
    )(x)


GROUP = 8


def _sc1_body(n_acc, r_acc, ch,
              h_hbm, srci_hbm, dsti_hbm, eti_hbm,
              out_s, out_r,
              acc_s, acc_r,
              srci, dsti, eti, rows, sem):
    cid = lax.axis_index("c")
    sid = lax.axis_index("s")
    wid = cid * NS + sid
    d = rows.shape[1]

    def fill_row(i, _):
        for j in range(d // 16):
            rows[i, pl.ds(j * 16, 16)] = jnp.zeros((16,), F32)
        return 0
    lax.fori_loop(0, CHUNK, fill_row, 0)

    n_per = n_acc // NS
    r_per = r_acc // NS
    for k in range(n_per // CHUNK):
        base = sid * n_per + k * CHUNK
        pltpu.sync_copy(rows, acc_s.at[pl.ds(base, CHUNK)])
    pltpu.sync_copy(rows.at[pl.ds(0, r_per)], acc_r.at[pl.ds(sid * r_per, r_per)])
    plsc.subcore_barrier()

    def group(g, _):
        base = pl.multiple_of(wid * ch + g * GROUP, GROUP)
        pltpu.sync_copy(srci_hbm.at[pl.ds(base, GROUP)], srci)
        pltpu.sync_copy(dsti_hbm.at[pl.ds(base, GROUP)], dsti)
        pltpu.sync_copy(eti_hbm.at[pl.ds(base, GROUP)], eti)

        def step(c, _):
            pltpu.async_copy(h_hbm.at[srci.at[c]], rows, sem).wait()
            pltpu.sync_copy(rows, acc_s.at[dsti.at[c]], add=True)
            pltpu.sync_copy(rows, acc_r.at[eti.at[c]], add=True)
            return 0
        lax.fori_loop(0, GROUP, step, 0)
        return 0
    lax.fori_loop(0, ch // GROUP, group, 0)
    plsc.subcore_barrier()

    for k in range(n_per // CHUNK):
        base = sid * n_per + k * CHUNK
        pltpu.sync_copy(acc_s.at[pl.ds(base, CHUNK)], out_s.at[cid, pl.ds(base, CHUNK)])
    pltpu.sync_copy(acc_r.at[pl.ds(sid * r_per, r_per)], out_r.at[cid, pl.ds(sid * r_per, r_per)])


def _sc_pass1(h, srci, dsti, eti, n_acc, r_acc):
    d = h.shape[1]
    ch = srci.shape[0] // NW
    mesh = plsc.VectorSubcoreMesh(core_axis_name="c", subcore_axis_name="s",
                                  num_cores=NC, num_subcores=NS)
    kern = pl.kernel(
        functools.partial(_sc1_body, n_acc, r_acc, ch),
        out_type=(
            jax.ShapeDtypeStruct((NC, n_acc, d), F32),
            jax.ShapeDtypeStruct((NC, r_acc, d), F32),
        ),
        mesh=mesh,
        scratch_types=[
            pltpu.VMEM_SHARED((n_acc, d), F32),
            pltpu.VMEM_SHARED((r_acc, d), F32),
            pltpu.VMEM((GROUP, CHUNK), jnp.int32),
            pltpu.VMEM((GROUP, CHUNK), jnp.int32),
            pltpu.VMEM((GROUP, CHUNK), jnp.int32),
            pltpu.VMEM((CHUNK, d), F32),
            pltpu.SemaphoreType.DMA,
        ],
    )
    return kern(h, srci, dsti, eti)


CGROUP = 512


def _scc_body(n_acc, hsize, epw,
              dsti_hbm, eti_hbm, out_h,
              idx_d, idx_e, hist):
    cid = lax.axis_index("c")
    sid = lax.axis_index("s")
    wid = cid * NS + sid

    def fill(i, _):
        hist[pl.ds(i * 16, 16)] = jnp.zeros((16,), F32)
        return 0
    lax.fori_loop(0, hsize // 16, fill, 0)

    def group(g, _):
        base = pl.multiple_of(wid * epw + g * CGROUP, 8)
        pltpu.sync_copy(dsti_hbm.at[pl.ds(base, CGROUP)], idx_d)
        pltpu.sync_copy(eti_hbm.at[pl.ds(base, CGROUP)], idx_e)

        def step(k, _):
            dv = idx_d[pl.ds(k * 16, 16)]
            cnts, last = plsc.scan_count(dv)
            plsc.addupdate_scatter(hist, [dv], cnts.astype(F32), mask=last)
            ev = idx_e[pl.ds(k * 16, 16)] + n_acc
            cnts2, last2 = plsc.scan_count(ev)
            plsc.addupdate_scatter(hist, [ev], cnts2.astype(F32), mask=last2)
            return 0
        lax.fori_loop(0, CGROUP // 16, step, 0)
        return 0
    lax.fori_loop(0, epw // CGROUP, group, 0)

    pltpu.sync_copy(hist, out_h.at[pl.ds(wid * hsize, hsize)])


def _sc_counts(dsti_flat, eti_flat, n_acc, r_acc):
    ep = dsti_flat.shape[0]
    epw = ep // NW
    hsize = n_acc + r_acc
    assert epw % CGROUP == 0 and hsize % 16 == 0 and (hsize % 8 == 0)
    mesh = plsc.VectorSubcoreMesh(core_axis_name="c", subcore_axis_name="s",
                                  num_cores=NC, num_subcores=NS)
    kern = pl.kernel(
        functools.partial(_scc_body, n_acc, hsize, epw),
        out_type=jax.ShapeDtypeStruct((NW * hsize,), F32),
        mesh=mesh,
        scratch_types=[
            pltpu.VMEM((CGROUP,), jnp.int32),
            pltpu.VMEM((CGROUP,), jnp.int32),
            pltpu.VMEM((hsize,), F32),
        ],
        compiler_params=pltpu.CompilerParams(needs_layout_passes=False),
    )
    return kern(dsti_flat, eti_flat)


def _sc2_body(n_acc, ch,
              h0_hbm, dsti_hbm, eti_hbm, out_t,
              acc_t, h0_sp, dsti, eti, rows, sem):
    cid = lax.axis_index("c")
    sid = lax.axis_index("s")
    wid = cid * NS + sid
    d = rows.shape[1]

    def fill_row(i, _):
        for j in range(d // 16):
            rows[i, pl.ds(j * 16, 16)] = jnp.zeros((16,), F32)
        return 0
    lax.fori_loop(0, CHUNK, fill_row, 0)

    n_per = n_acc // NS
    for k in range(n_per // CHUNK):
        base = sid * n_per + k * CHUNK
        pltpu.sync_copy(rows, acc_t.at[pl.ds(base, CHUNK)])
    @pl.when(sid == 0)
    def _():
        pltpu.sync_copy(h0_hbm, h0_sp)
    plsc.subcore_barrier()

    def group(g, _):
        base = pl.multiple_of(wid * ch + g * GROUP, GROUP)
        pltpu.sync_copy(dsti_hbm.at[pl.ds(base, GROUP)], dsti)
        pltpu.sync_copy(eti_hbm.at[pl.ds(base, GROUP)], eti)

        def step(c, _):
            pltpu.async_copy(h0_sp.at[eti.at[c]], rows, sem).wait()
            pltpu.sync_copy(rows, acc_t.at[dsti.at[c]], add=True)
            return 0
        lax.fori_loop(0, GROUP, step, 0)
        return 0
    lax.fori_loop(0, ch // GROUP, group, 0)
    plsc.subcore_barrier()

    for k in range(n_per // CHUNK):
        base = sid * n_per + k * CHUNK
        pltpu.sync_copy(acc_t.at[pl.ds(base, CHUNK)], out_t.at[cid, pl.ds(base, CHUNK)])


def _sc_pass2(h0, dsti, eti, n_acc):
    r_acc, d = h0.shape
    ch = dsti.shape[0] // NW
    mesh = plsc.VectorSubcoreMesh(core_axis_name="c", subcore_axis_name="s",
                                  num_cores=NC, num_subcores=NS)
    kern = pl.kernel(
        functools.partial(_sc2_body, n_acc, ch),
        out_type=jax.ShapeDtypeStruct((NC, n_acc, d), F32),
        mesh=mesh,
        scratch_types=[
            pltpu.VMEM_SHARED((n_acc, d), F32),
            pltpu.VMEM_SHARED((r_acc, d), F32),
            pltpu.VMEM((GROUP, CHUNK), jnp.int32),
            pltpu.VMEM((GROUP, CHUNK), jnp.int32),
            pltpu.VMEM((CHUNK, d), F32),
            pltpu.SemaphoreType.DMA,
        ],
    )
    return kern(h0, dsti, eti)


def _tc_gru_body(relp_ref, cntp_ref, emb_ref, wih_ref, whh_ref, bih_ref, bhh_ref, o_ref):
    d = emb_ref.shape[1]
    rel_sum = relp_ref[0] + relp_ref[1]
    cnt = jnp.sum(cntp_ref[...], axis=0)
    rel_mean = rel_sum / jnp.maximum(cnt, 1.0)
    emb = emb_ref[...]
    x = jnp.concatenate([emb, rel_mean], axis=1)
    gi = lax.dot_general(x, wih_ref[...], (((1,), (1,)), ((), ())),
                         preferred_element_type=F32) + bih_ref[...]
    gh = lax.dot_general(emb, whh_ref[...], (((1,), (1,)), ((), ())),
                         preferred_element_type=F32) + bhh_ref[...]
    i_r, i_z, i_n = gi[:, :d], gi[:, d:2 * d], gi[:, 2 * d:]
    h_r, h_z, h_n = gh[:, :d], gh[:, d:2 * d], gh[:, 2 * d:]
    r = jax.nn.sigmoid(i_r + h_r)
    z = jax.nn.sigmoid(i_z + h_z)
    n = jnp.tanh(i_n + r * h_n)
    h0 = (1.0 - z) * n + z * emb
    o_ref[...] = _l2norm(h0)


def _tc_gru(rel_p, cnt_p, emb_p, w_ih, w_hh, b_ih, b_hh):
    r_acc, d = emb_p.shape
    return pl.pallas_call(
        _tc_gru_body,
        out_shape=jax.ShapeDtypeStruct((r_acc, d), F32),
    )(rel_p, cnt_p, emb_p, w_ih, w_hh, b_ih, b_hh)


def _tc_final_body(h_ref, s_ref, t_ref, d_ref, wn_ref, lw_ref, tgw_ref, tgb_ref, o_ref):
    u = (s_ref[0] - t_ref[0]) + (s_ref[1] - t_ref[1])
    deg = jnp.sum(d_ref[...], axis=0)
    agg = jnp.dot(u, wn_ref[...], preferred_element_type=F32) / jnp.maximum(deg, 1.0)
    h = h_ref[...]
    cur = agg + jnp.dot(h, lw_ref[...], preferred_element_type=F32)
    slope = (1.0 / 8.0 + 1.0 / 3.0) / 2.0
    cur = jnp.where(cur >= 0, cur, slope * cur)
    cur = _l2norm(cur)
    gate = jax.nn.sigmoid(jnp.dot(cur, tgw_ref[...], preferred_element_type=F32)
                          + tgb_ref[...])
    o_ref[...] = gate * cur + (1.0 - gate) * h


def _tc_final(h, s_p, t_p, d_p, w_n, l_w, tg_w, tg_b, block=1024):
    n_acc, d = h.shape
    grid = n_acc // block
    return pl.pallas_call(
        _tc_final_body,
        grid=(grid,),
        in_specs=[
            pl.BlockSpec((block, d), lambda i: (i, 0)),
            pl.BlockSpec((NC, block, d), lambda i: (0, i, 0)),
            pl.BlockSpec((NC, block, d), lambda i: (0, i, 0)),
            pl.BlockSpec((NW, block, 1), lambda i: (0, i, 0)),
            pl.BlockSpec((d, d), lambda i: (0, 0)),
            pl.BlockSpec((d, d), lambda i: (0, 0)),
            pl.BlockSpec((d, d), lambda i: (0, 0)),
            pl.BlockSpec((1, d), lambda i: (0, 0)),
        ],
        out_specs=pl.BlockSpec((block, d), lambda i: (i, 0)),
        out_shape=jax.ShapeDtypeStruct((n_acc, d), F32),
    )(h, s_p, t_p, d_p, w_n, l_w, tg_w, tg_b)


def kernel(edge_index, edge_type, dynamic_emb, emb_rel, W_ih, W_hh, b_ih, b_hh,
           w_neighbor, loop_weight, time_gate_weight, time_gate_bias):
    n, d = dynamic_emb.shape
    r = emb_rel.shape[0]
    e = edge_type.shape[0]

    ep = ((e + NW * CHUNK * 8 - 1) // (NW * CHUNK * 8)) * (NW * CHUNK * 8)
    pad = ep - e
    n_acc = ((n + NS * CHUNK) // (NS * CHUNK)) * (NS * CHUNK)
    r_acc = ((r + NS - 1) // NS + 1) * NS
    r_acc = ((r_acc + NS * 8 - 1) // (NS * 8)) * (NS * 8)

    src = edge_index[0]
    dst = edge_index[1]
    srci = jnp.concatenate([src, jnp.zeros((pad,), jnp.int32)]).reshape(-1, CHUNK)
    dsti = jnp.concatenate([dst, jnp.full((pad,), n, jnp.int32)]).reshape(-1, CHUNK)
    eti = jnp.concatenate([edge_type, jnp.full((pad,), r, jnp.int32)]).reshape(-1, CHUNK)

    de_p = jnp.zeros((n_acc, d), F32).at[:n].set(dynamic_emb)
    h_pad = _tc_norm(de_p)

    s_p, rel_p = _sc_pass1(h_pad, srci, dsti, eti, n_acc, r_acc)
    hist_flat = _sc_counts(dsti.reshape(-1), eti.reshape(-1), n_acc, r_acc)
    hist_flat = hist_flat.reshape(NW, n_acc + r_acc)
    deg_p = hist_flat[:, :n_acc].reshape(NW, n_acc, 1)
    cnt_p = hist_flat[:, n_acc:].reshape(NW, r_acc, 1)

    emb_p = jnp.zeros((r_acc, d), F32).at[:r].set(emb_rel)
    h0 = _tc_gru(rel_p, cnt_p, emb_p, W_ih, W_hh,
                 b_ih.reshape(1, -1), b_hh.reshape(1, -1))

    t_p = _sc_pass2(h0, dsti, eti, n_acc)

    h_new = _tc_final(h_pad, s_p, t_p, deg_p, w_neighbor, loop_weight,
                      time_gate_weight, time_gate_bias.reshape(1, -1))
    return h_new[:n]


# speedup vs baseline: 4.1420x; 1.1299x over previous
import functools

import jax
import jax.numpy as jnp
from jax import lax
from jax.experimental import pallas as pl
from jax.experimental.pallas import tpu as pltpu
from jax.experimental.pallas import tpu_sc as plsc

F32 = jnp.float32

NC = 2
NS = 16
NW = NC * NS
CHUNK = 64


def _l2norm(x):
    n = jnp.sqrt(jnp.sum(x * x, axis=-1, keepdims=True))
    return x / jnp.clip(n, 1e-12, None)


def _tc_norm_body(x_ref, o_ref):
    o_ref[...] = _l2norm(x_ref[...])


def _tc_norm(x, block=1024):
    m, d = x.shape
    grid = m // block
    return pl.pallas_call(
        _tc_norm_body,
        grid=(grid,),
        in_specs=[pl.BlockSpec((block, d), lambda i: (i, 0))],
        out_specs=pl.BlockSpec((block, d), lambda i: (i, 0)),
        out_shape=jax.ShapeDtypeStruct((m, d), F32),
    )(x)


GROUP = 8


def _sc1_body(n_acc, r_acc, ch,
              h_hbm, srci_hbm, dsti_hbm, eti_hbm,
              out_s, out_r,
              acc_s, acc_r,
              srci, dsti, eti, rows, rows2, sem, sem2):
    cid = lax.axis_index("c")
    sid = lax.axis_index("s")
    wid = cid * NS + sid
    d = rows.shape[1]

    def fill_row(i, _):
        for j in range(d // 16):
            rows[i, pl.ds(j * 16, 16)] = jnp.zeros((16,), F32)
        return 0
    lax.fori_loop(0, CHUNK, fill_row, 0)

    n_per = n_acc // NS
    r_per = r_acc // NS
    for k in range(n_per // CHUNK):
        base = sid * n_per + k * CHUNK
        pltpu.sync_copy(rows, acc_s.at[pl.ds(base, CHUNK)])
    pltpu.sync_copy(rows.at[pl.ds(0, r_per)], acc_r.at[pl.ds(sid * r_per, r_per)])
    plsc.subcore_barrier()

    def group(g, _):
        base = pl.multiple_of(wid * ch + g * GROUP, GROUP)
        pltpu.sync_copy(srci_hbm.at[pl.ds(base, GROUP)], srci)
        pltpu.sync_copy(dsti_hbm.at[pl.ds(base, GROUP)], dsti)
        pltpu.sync_copy(eti_hbm.at[pl.ds(base, GROUP)], eti)

        bufs = (rows, rows2)
        sems = (sem, sem2)
        cps = [None, None]
        cps[0] = pltpu.async_copy(h_hbm.at[srci.at[0]], bufs[0], sems[0])
        for c in range(GROUP):
            if c + 1 < GROUP:
                cps[(c + 1) % 2] = pltpu.async_copy(
                    h_hbm.at[srci.at[c + 1]], bufs[(c + 1) % 2], sems[(c + 1) % 2])
            cps[c % 2].wait()
            pltpu.sync_copy(bufs[c % 2], acc_s.at[dsti.at[c]], add=True)
            pltpu.sync_copy(bufs[c % 2], acc_r.at[eti.at[c]], add=True)
        return 0
    lax.fori_loop(0, ch // GROUP, group, 0)
    plsc.subcore_barrier()

    for k in range(n_per // CHUNK):
        base = sid * n_per + k * CHUNK
        pltpu.sync_copy(acc_s.at[pl.ds(base, CHUNK)], out_s.at[cid, pl.ds(base, CHUNK)])
    pltpu.sync_copy(acc_r.at[pl.ds(sid * r_per, r_per)], out_r.at[cid, pl.ds(sid * r_per, r_per)])


def _sc_pass1(h, srci, dsti, eti, n_acc, r_acc):
    d = h.shape[1]
    ch = srci.shape[0] // NW
    mesh = plsc.VectorSubcoreMesh(core_axis_name="c", subcore_axis_name="s",
                                  num_cores=NC, num_subcores=NS)
    kern = pl.kernel(
        functools.partial(_sc1_body, n_acc, r_acc, ch),
        out_type=(
            jax.ShapeDtypeStruct((NC, n_acc, d), F32),
            jax.ShapeDtypeStruct((NC, r_acc, d), F32),
        ),
        mesh=mesh,
        scratch_types=[
            pltpu.VMEM_SHARED((n_acc, d), F32),
            pltpu.VMEM_SHARED((r_acc, d), F32),
            pltpu.VMEM((GROUP, CHUNK), jnp.int32),
            pltpu.VMEM((GROUP, CHUNK), jnp.int32),
            pltpu.VMEM((GROUP, CHUNK), jnp.int32),
            pltpu.VMEM((CHUNK, d), F32),
            pltpu.VMEM((CHUNK, d), F32),
            pltpu.SemaphoreType.DMA,
            pltpu.SemaphoreType.DMA,
        ],
    )
    return kern(h, srci, dsti, eti)


CGROUP = 512


def _scc_body(n_acc, hsize, epw,
              dsti_hbm, eti_hbm, out_h,
              idx_d, idx_e, hist):
    cid = lax.axis_index("c")
    sid = lax.axis_index("s")
    wid = cid * NS + sid

    def fill(i, _):
        hist[pl.ds(i * 16, 16)] = jnp.zeros((16,), F32)
        return 0
    lax.fori_loop(0, hsize // 16, fill, 0)

    def group(g, _):
        base = pl.multiple_of(wid * epw + g * CGROUP, 8)
        pltpu.sync_copy(dsti_hbm.at[pl.ds(base, CGROUP)], idx_d)
        pltpu.sync_copy(eti_hbm.at[pl.ds(base, CGROUP)], idx_e)

        def step(k, _):
            dv = idx_d[pl.ds(k * 16, 16)]
            cnts, last = plsc.scan_count(dv)
            plsc.addupdate_scatter(hist, [dv], cnts.astype(F32), mask=last)
            ev = idx_e[pl.ds(k * 16, 16)] + n_acc
            cnts2, last2 = plsc.scan_count(ev)
            plsc.addupdate_scatter(hist, [ev], cnts2.astype(F32), mask=last2)
            return 0
        lax.fori_loop(0, CGROUP // 16, step, 0)
        return 0
    lax.fori_loop(0, epw // CGROUP, group, 0)

    pltpu.sync_copy(hist, out_h.at[pl.ds(wid * hsize, hsize)])


def _sc_counts(dsti_flat, eti_flat, n_acc, r_acc):
    ep = dsti_flat.shape[0]
    epw = ep // NW
    hsize = n_acc + r_acc
    assert epw % CGROUP == 0 and hsize % 16 == 0 and (hsize % 8 == 0)
    mesh = plsc.VectorSubcoreMesh(core_axis_name="c", subcore_axis_name="s",
                                  num_cores=NC, num_subcores=NS)
    kern = pl.kernel(
        functools.partial(_scc_body, n_acc, hsize, epw),
        out_type=jax.ShapeDtypeStruct((NW * hsize,), F32),
        mesh=mesh,
        scratch_types=[
            pltpu.VMEM((CGROUP,), jnp.int32),
            pltpu.VMEM((CGROUP,), jnp.int32),
            pltpu.VMEM((hsize,), F32),
        ],
        compiler_params=pltpu.CompilerParams(needs_layout_passes=False),
    )
    return kern(dsti_flat, eti_flat)


def _sc2_body(n_acc, ch,
              h0_hbm, dsti_hbm, eti_hbm, out_t,
              acc_t, h0_sp, dsti, eti, rows, sem):
    cid = lax.axis_index("c")
    sid = lax.axis_index("s")
    wid = cid * NS + sid
    d = rows.shape[1]

    def fill_row(i, _):
        for j in range(d // 16):
            rows[i, pl.ds(j * 16, 16)] = jnp.zeros((16,), F32)
        return 0
    lax.fori_loop(0, CHUNK, fill_row, 0)

    n_per = n_acc // NS
    for k in range(n_per // CHUNK):
        base = sid * n_per + k * CHUNK
        pltpu.sync_copy(rows, acc_t.at[pl.ds(base, CHUNK)])
    @pl.when(sid == 0)
    def _():
        pltpu.sync_copy(h0_hbm, h0_sp)
    plsc.subcore_barrier()

    def group(g, _):
        base = pl.multiple_of(wid * ch + g * GROUP, GROUP)
        pltpu.sync_copy(dsti_hbm.at[pl.ds(base, GROUP)], dsti)
        pltpu.sync_copy(eti_hbm.at[pl.ds(base, GROUP)], eti)

        def step(c, _):
            pltpu.async_copy(h0_sp.at[eti.at[c]], rows, sem).wait()
            pltpu.sync_copy(rows, acc_t.at[dsti.at[c]], add=True)
            return 0
        lax.fori_loop(0, GROUP, step, 0)
        return 0
    lax.fori_loop(0, ch // GROUP, group, 0)
    plsc.subcore_barrier()

    for k in range(n_per // CHUNK):
        base = sid * n_per + k * CHUNK
        pltpu.sync_copy(acc_t.at[pl.ds(base, CHUNK)], out_t.at[cid, pl.ds(base, CHUNK)])


def _sc_pass2(h0, dsti, eti, n_acc):
    r_acc, d = h0.shape
    ch = dsti.shape[0] // NW
    mesh = plsc.VectorSubcoreMesh(core_axis_name="c", subcore_axis_name="s",
                                  num_cores=NC, num_subcores=NS)
    kern = pl.kernel(
        functools.partial(_sc2_body, n_acc, ch),
        out_type=jax.ShapeDtypeStruct((NC, n_acc, d), F32),
        mesh=mesh,
        scratch_types=[
            pltpu.VMEM_SHARED((n_acc, d), F32),
            pltpu.VMEM_SHARED((r_acc, d), F32),
            pltpu.VMEM((GROUP, CHUNK), jnp.int32),
            pltpu.VMEM((GROUP, CHUNK), jnp.int32),
            pltpu.VMEM((CHUNK, d), F32),
            pltpu.SemaphoreType.DMA,
        ],
    )
    return kern(h0, dsti, eti)


def _tc_gru_body(relp_ref, cntp_ref, emb_ref, wih_ref, whh_ref, bih_ref, bhh_ref, o_ref):
    d = emb_ref.shape[1]
    rel_sum = relp_ref[0] + relp_ref[1]
    cnt = jnp.sum(cntp_ref[...], axis=0)
    rel_mean = rel_sum / jnp.maximum(cnt, 1.0)
    emb = emb_ref[...]
    x = jnp.concatenate([emb, rel_mean], axis=1)
    gi = lax.dot_general(x, wih_ref[...], (((1,), (1,)), ((), ())),
                         preferred_element_type=F32) + bih_ref[...]
    gh = lax.dot_general(emb, whh_ref[...], (((1,), (1,)), ((), ())),
                         preferred_element_type=F32) + bhh_ref[...]
    i_r, i_z, i_n = gi[:, :d], gi[:, d:2 * d], gi[:, 2 * d:]
    h_r, h_z, h_n = gh[:, :d], gh[:, d:2 * d], gh[:, 2 * d:]
    r = jax.nn.sigmoid(i_r + h_r)
    z = jax.nn.sigmoid(i_z + h_z)
    n = jnp.tanh(i_n + r * h_n)
    h0 = (1.0 - z) * n + z * emb
    o_ref[...] = _l2norm(h0)


def _tc_gru(rel_p, cnt_p, emb_p, w_ih, w_hh, b_ih, b_hh):
    r_acc, d = emb_p.shape
    return pl.pallas_call(
        _tc_gru_body,
        out_shape=jax.ShapeDtypeStruct((r_acc, d), F32),
    )(rel_p, cnt_p, emb_p, w_ih, w_hh, b_ih, b_hh)


def _tc_final_body(h_ref, s_ref, t_ref, d_ref, wn_ref, lw_ref, tgw_ref, tgb_ref, o_ref):
    u = (s_ref[0] - t_ref[0]) + (s_ref[1] - t_ref[1])
    deg = jnp.sum(d_ref[...], axis=0)
    agg = jnp.dot(u, wn_ref[...], preferred_element_type=F32) / jnp.maximum(deg, 1.0)
    h = h_ref[...]
    cur = agg + jnp.dot(h, lw_ref[...], preferred_element_type=F32)
    slope = (1.0 / 8.0 + 1.0 / 3.0) / 2.0
    cur = jnp.where(cur >= 0, cur, slope * cur)
    cur = _l2norm(cur)
    gate = jax.nn.sigmoid(jnp.dot(cur, tgw_ref[...], preferred_element_type=F32)
                          + tgb_ref[...])
    o_ref[...] = gate * cur + (1.0 - gate) * h


def _tc_final(h, s_p, t_p, d_p, w_n, l_w, tg_w, tg_b, block=1024):
    n_acc, d = h.shape
    grid = n_acc // block
    return pl.pallas_call(
        _tc_final_body,
        grid=(grid,),
        in_specs=[
            pl.BlockSpec((block, d), lambda i: (i, 0)),
            pl.BlockSpec((NC, block, d), lambda i: (0, i, 0)),
            pl.BlockSpec((NC, block, d), lambda i: (0, i, 0)),
            pl.BlockSpec((NW, block, 1), lambda i: (0, i, 0)),
            pl.BlockSpec((d, d), lambda i: (0, 0)),
            pl.BlockSpec((d, d), lambda i: (0, 0)),
            pl.BlockSpec((d, d), lambda i: (0, 0)),
            pl.BlockSpec((1, d), lambda i: (0, 0)),
        ],
        out_specs=pl.BlockSpec((block, d), lambda i: (i, 0)),
        out_shape=jax.ShapeDtypeStruct((n_acc, d), F32),
    )(h, s_p, t_p, d_p, w_n, l_w, tg_w, tg_b)


def kernel(edge_index, edge_type, dynamic_emb, emb_rel, W_ih, W_hh, b_ih, b_hh,
           w_neighbor, loop_weight, time_gate_weight, time_gate_bias):
    n, d = dynamic_emb.shape
    r = emb_rel.shape[0]
    e = edge_type.shape[0]

    ep = ((e + NW * CHUNK * 8 - 1) // (NW * CHUNK * 8)) * (NW * CHUNK * 8)
    pad = ep - e
    n_acc = ((n + NS * CHUNK) // (NS * CHUNK)) * (NS * CHUNK)
    r_acc = ((r + NS - 1) // NS + 1) * NS
    r_acc = ((r_acc + NS * 8 - 1) // (NS * 8)) * (NS * 8)

    src = edge_index[0]
    dst = edge_index[1]
    srci = jnp.concatenate([src, jnp.zeros((pad,), jnp.int32)]).reshape(-1, CHUNK)
    dsti = jnp.concatenate([dst, jnp.full((pad,), n, jnp.int32)]).reshape(-1, CHUNK)
    eti = jnp.concatenate([edge_type, jnp.full((pad,), r, jnp.int32)]).reshape(-1, CHUNK)

    de_p = jnp.zeros((n_acc, d), F32).at[:n].set(dynamic_emb)
    h_pad = _tc_norm(de_p)

    s_p, rel_p = _sc_pass1(h_pad, srci, dsti, eti, n_acc, r_acc)
    hist_flat = _sc_counts(dsti.reshape(-1), eti.reshape(-1), n_acc, r_acc)
    hist_flat = hist_flat.reshape(NW, n_acc + r_acc)
    deg_p = hist_flat[:, :n_acc].reshape(NW, n_acc, 1)
    cnt_p = hist_flat[:, n_acc:].reshape(NW, r_acc, 1)

    emb_p = jnp.zeros((r_acc, d), F32).at[:r].set(emb_rel)
    h0 = _tc_gru(rel_p, cnt_p, emb_p, W_ih, W_hh,
                 b_ih.reshape(1, -1), b_hh.reshape(1, -1))

    t_p = _sc_pass2(h0, dsti, eti, n_acc)

    h_new = _tc_final(h_pad, s_p, t_p, deg_p, w_neighbor, loop_weight,
                      time_gate_weight, time_gate_bias.reshape(1, -1))
    return h_new[:n]


# speedup vs baseline: 4.4084x; 1.0643x over previous
import functools

import jax
import jax.numpy as jnp
from jax import lax
from jax.experimental import pallas as pl
from jax.experimental.pallas import tpu as pltpu
from jax.experimental.pallas import tpu_sc as plsc

F32 = jnp.float32

NC = 2
NS = 16
NW = NC * NS
CHUNK = 64


def _l2norm(x):
    n = jnp.sqrt(jnp.sum(x * x, axis=-1, keepdims=True))
    return x / jnp.clip(n, 1e-12, None)


def _tc_norm_body(x_ref, o_ref):
    o_ref[...] = _l2norm(x_ref[...])


def _tc_norm(x, block=1024):
    m, d = x.shape
    grid = m // block
    return pl.pallas_call(
        _tc_norm_body,
        grid=(grid,),
        in_specs=[pl.BlockSpec((block, d), lambda i: (i, 0))],
        out_specs=pl.BlockSpec((block, d), lambda i: (i, 0)),
        out_shape=jax.ShapeDtypeStruct((m, d), F32),
    )(x)


GROUP = 8


def _sc1_body(n_acc, r_acc, ch,
              h_hbm, srci_hbm, dsti_hbm, eti_hbm,
              out_s, out_r,
              acc_s, acc_r,
              srci, dsti, eti, rows, rows2, rows3, sem, sem2, sem3,
              ssem, ssem2, ssem3):
    cid = lax.axis_index("c")
    sid = lax.axis_index("s")
    wid = cid * NS + sid
    d = rows.shape[1]

    def fill_row(i, _):
        for j in range(d // 16):
            rows[i, pl.ds(j * 16, 16)] = jnp.zeros((16,), F32)
        return 0
    lax.fori_loop(0, CHUNK, fill_row, 0)

    n_per = n_acc // NS
    r_per = r_acc // NS
    for k in range(n_per // CHUNK):
        base = sid * n_per + k * CHUNK
        pltpu.sync_copy(rows, acc_s.at[pl.ds(base, CHUNK)])
    pltpu.sync_copy(rows.at[pl.ds(0, r_per)], acc_r.at[pl.ds(sid * r_per, r_per)])
    plsc.subcore_barrier()

    def group(g, _):
        base = pl.multiple_of(wid * ch + g * GROUP, GROUP)
        pltpu.sync_copy(srci_hbm.at[pl.ds(base, GROUP)], srci)
        pltpu.sync_copy(dsti_hbm.at[pl.ds(base, GROUP)], dsti)
        pltpu.sync_copy(eti_hbm.at[pl.ds(base, GROUP)], eti)

        bufs = (rows, rows2, rows3)
        gsems = (sem, sem2, sem3)
        ssems = (ssem, ssem2, ssem3)
        gd = {}
        sd = {}
        gd[0] = pltpu.async_copy(h_hbm.at[srci.at[0]], bufs[0], gsems[0])
        for c in range(GROUP):
            b = c % 3
            if c + 1 < GROUP:
                bn = (c + 1) % 3
                if c - 2 >= 0:
                    sd[c - 2][0].wait()
                    sd[c - 2][1].wait()
                gd[c + 1] = pltpu.async_copy(
                    h_hbm.at[srci.at[c + 1]], bufs[bn], gsems[bn])
            gd[c].wait()
            sd[c] = (
                pltpu.async_copy(bufs[b], acc_s.at[dsti.at[c]], ssems[b], add=True),
                pltpu.async_copy(bufs[b], acc_r.at[eti.at[c]], ssems[b], add=True),
            )
        for c in (GROUP - 3, GROUP - 2, GROUP - 1):
            sd[c][0].wait()
            sd[c][1].wait()
        return 0
    lax.fori_loop(0, ch // GROUP, group, 0)
    plsc.subcore_barrier()

    for k in range(n_per // CHUNK):
        base = sid * n_per + k * CHUNK
        pltpu.sync_copy(acc_s.at[pl.ds(base, CHUNK)], out_s.at[cid, pl.ds(base, CHUNK)])
    pltpu.sync_copy(acc_r.at[pl.ds(sid * r_per, r_per)], out_r.at[cid, pl.ds(sid * r_per, r_per)])


def _sc_pass1(h, srci, dsti, eti, n_acc, r_acc):
    d = h.shape[1]
    ch = srci.shape[0] // NW
    mesh = plsc.VectorSubcoreMesh(core_axis_name="c", subcore_axis_name="s",
                                  num_cores=NC, num_subcores=NS)
    kern = pl.kernel(
        functools.partial(_sc1_body, n_acc, r_acc, ch),
        out_type=(
            jax.ShapeDtypeStruct((NC, n_acc, d), F32),
            jax.ShapeDtypeStruct((NC, r_acc, d), F32),
        ),
        mesh=mesh,
        scratch_types=[
            pltpu.VMEM_SHARED((n_acc, d), F32),
            pltpu.VMEM_SHARED((r_acc, d), F32),
            pltpu.VMEM((GROUP, CHUNK), jnp.int32),
            pltpu.VMEM((GROUP, CHUNK), jnp.int32),
            pltpu.VMEM((GROUP, CHUNK), jnp.int32),
            pltpu.VMEM((CHUNK, d), F32),
            pltpu.VMEM((CHUNK, d), F32),
            pltpu.VMEM((CHUNK, d), F32),
            pltpu.SemaphoreType.DMA,
            pltpu.SemaphoreType.DMA,
            pltpu.SemaphoreType.DMA,
            pltpu.SemaphoreType.DMA,
            pltpu.SemaphoreType.DMA,
            pltpu.SemaphoreType.DMA,
        ],
    )
    return kern(h, srci, dsti, eti)


CGROUP = 512


def _scc_body(n_acc, hsize, epw,
              dsti_hbm, eti_hbm, out_h,
              idx_d, idx_e, hist):
    cid = lax.axis_index("c")
    sid = lax.axis_index("s")
    wid = cid * NS + sid

    def fill(i, _):
        hist[pl.ds(i * 16, 16)] = jnp.zeros((16,), F32)
        return 0
    lax.fori_loop(0, hsize // 16, fill, 0)

    def group(g, _):
        base = pl.multiple_of(wid * epw + g * CGROUP, 8)
        pltpu.sync_copy(dsti_hbm.at[pl.ds(base, CGROUP)], idx_d)
        pltpu.sync_copy(eti_hbm.at[pl.ds(base, CGROUP)], idx_e)

        def step(k, _):
            dv = idx_d[pl.ds(k * 16, 16)]
            cnts, last = plsc.scan_count(dv)
            plsc.addupdate_scatter(hist, [dv], cnts.astype(F32), mask=last)
            ev = idx_e[pl.ds(k * 16, 16)] + n_acc
            cnts2, last2 = plsc.scan_count(ev)
            plsc.addupdate_scatter(hist, [ev], cnts2.astype(F32), mask=last2)
            return 0
        lax.fori_loop(0, CGROUP // 16, step, 0)
        return 0
    lax.fori_loop(0, epw // CGROUP, group, 0)

    pltpu.sync_copy(hist, out_h.at[pl.ds(wid * hsize, hsize)])


def _sc_counts(dsti_flat, eti_flat, n_acc, r_acc):
    ep = dsti_flat.shape[0]
    epw = ep // NW
    hsize = n_acc + r_acc
    assert epw % CGROUP == 0 and hsize % 16 == 0 and (hsize % 8 == 0)
    mesh = plsc.VectorSubcoreMesh(core_axis_name="c", subcore_axis_name="s",
                                  num_cores=NC, num_subcores=NS)
    kern = pl.kernel(
        functools.partial(_scc_body, n_acc, hsize, epw),
        out_type=jax.ShapeDtypeStruct((NW * hsize,), F32),
        mesh=mesh,
        scratch_types=[
            pltpu.VMEM((CGROUP,), jnp.int32),
            pltpu.VMEM((CGROUP,), jnp.int32),
            pltpu.VMEM((hsize,), F32),
        ],
        compiler_params=pltpu.CompilerParams(needs_layout_passes=False),
    )
    return kern(dsti_flat, eti_flat)


def _sc2_body(n_acc, ch,
              h0_hbm, dsti_hbm, eti_hbm, out_t,
              acc_t, h0_sp, dsti, eti, rows, rows2, rows3, sem, sem2, sem3,
              ssem, ssem2, ssem3):
    cid = lax.axis_index("c")
    sid = lax.axis_index("s")
    wid = cid * NS + sid
    d = rows.shape[1]

    def fill_row(i, _):
        for j in range(d // 16):
            rows[i, pl.ds(j * 16, 16)] = jnp.zeros((16,), F32)
        return 0
    lax.fori_loop(0, CHUNK, fill_row, 0)

    n_per = n_acc // NS
    for k in range(n_per // CHUNK):
        base = sid * n_per + k * CHUNK
        pltpu.sync_copy(rows, acc_t.at[pl.ds(base, CHUNK)])
    @pl.when(sid == 0)
    def _():
        pltpu.sync_copy(h0_hbm, h0_sp)
    plsc.subcore_barrier()

    def group(g, _):
        base = pl.multiple_of(wid * ch + g * GROUP, GROUP)
        pltpu.sync_copy(dsti_hbm.at[pl.ds(base, GROUP)], dsti)
        pltpu.sync_copy(eti_hbm.at[pl.ds(base, GROUP)], eti)

        bufs = (rows, rows2, rows3)
        gsems = (sem, sem2, sem3)
        ssems = (ssem, ssem2, ssem3)
        gd = {}
        sd = {}
        gd[0] = pltpu.async_copy(h0_sp.at[eti.at[0]], bufs[0], gsems[0])
        for c in range(GROUP):
            b = c % 3
            if c + 1 < GROUP:
                bn = (c + 1) % 3
                if c - 2 >= 0:
                    sd[c - 2].wait()
                gd[c + 1] = pltpu.async_copy(
                    h0_sp.at[eti.at[c + 1]], bufs[bn], gsems[bn])
            gd[c].wait()
            sd[c] = pltpu.async_copy(bufs[b], acc_t.at[dsti.at[c]], ssems[b], add=True)
        for c in (GROUP - 3, GROUP - 2, GROUP - 1):
            sd[c].wait()
        return 0
    lax.fori_loop(0, ch // GROUP, group, 0)
    plsc.subcore_barrier()

    for k in range(n_per // CHUNK):
        base = sid * n_per + k * CHUNK
        pltpu.sync_copy(acc_t.at[pl.ds(base, CHUNK)], out_t.at[cid, pl.ds(base, CHUNK)])


def _sc_pass2(h0, dsti, eti, n_acc):
    r_acc, d = h0.shape
    ch = dsti.shape[0] // NW
    mesh = plsc.VectorSubcoreMesh(core_axis_name="c", subcore_axis_name="s",
                                  num_cores=NC, num_subcores=NS)
    kern = pl.kernel(
        functools.partial(_sc2_body, n_acc, ch),
        out_type=jax.ShapeDtypeStruct((NC, n_acc, d), F32),
        mesh=mesh,
        scratch_types=[
            pltpu.VMEM_SHARED((n_acc, d), F32),
            pltpu.VMEM_SHARED((r_acc, d), F32),
            pltpu.VMEM((GROUP, CHUNK), jnp.int32),
            pltpu.VMEM((GROUP, CHUNK), jnp.int32),
            pltpu.VMEM((CHUNK, d), F32),
            pltpu.VMEM((CHUNK, d), F32),
            pltpu.VMEM((CHUNK, d), F32),
            pltpu.SemaphoreType.DMA,
            pltpu.SemaphoreType.DMA,
            pltpu.SemaphoreType.DMA,
            pltpu.SemaphoreType.DMA,
            pltpu.SemaphoreType.DMA,
            pltpu.SemaphoreType.DMA,
        ],
    )
    return kern(h0, dsti, eti)


def _tc_gru_body(relp_ref, cntp_ref, emb_ref, wih_ref, whh_ref, bih_ref, bhh_ref, o_ref):
    d = emb_ref.shape[1]
    rel_sum = relp_ref[0] + relp_ref[1]
    cnt = jnp.sum(cntp_ref[...], axis=0)
    rel_mean = rel_sum / jnp.maximum(cnt, 1.0)
    emb = emb_ref[...]
    x = jnp.concatenate([emb, rel_mean], axis=1)
    gi = lax.dot_general(x, wih_ref[...], (((1,), (1,)), ((), ())),
                         preferred_element_type=F32) + bih_ref[...]
    gh = lax.dot_general(emb, whh_ref[...], (((1,), (1,)), ((), ())),
                         preferred_element_type=F32) + bhh_ref[...]
    i_r, i_z, i_n = gi[:, :d], gi[:, d:2 * d], gi[:, 2 * d:]
    h_r, h_z, h_n = gh[:, :d], gh[:, d:2 * d], gh[:, 2 * d:]
    r = jax.nn.sigmoid(i_r + h_r)
    z = jax.nn.sigmoid(i_z + h_z)
    n = jnp.tanh(i_n + r * h_n)
    h0 = (1.0 - z) * n + z * emb
    o_ref[...] = _l2norm(h0)


def _tc_gru(rel_p, cnt_p, emb_p, w_ih, w_hh, b_ih, b_hh):
    r_acc, d = emb_p.shape
    return pl.pallas_call(
        _tc_gru_body,
        out_shape=jax.ShapeDtypeStruct((r_acc, d), F32),
    )(rel_p, cnt_p, emb_p, w_ih, w_hh, b_ih, b_hh)


def _tc_final_body(h_ref, s_ref, t_ref, d_ref, wn_ref, lw_ref, tgw_ref, tgb_ref, o_ref):
    u = (s_ref[0] - t_ref[0]) + (s_ref[1] - t_ref[1])
    deg = jnp.sum(d_ref[...], axis=0)
    agg = jnp.dot(u, wn_ref[...], preferred_element_type=F32) / jnp.maximum(deg, 1.0)
    h = h_ref[...]
    cur = agg + jnp.dot(h, lw_ref[...], preferred_element_type=F32)
    slope = (1.0 / 8.0 + 1.0 / 3.0) / 2.0
    cur = jnp.where(cur >= 0, cur, slope * cur)
    cur = _l2norm(cur)
    gate = jax.nn.sigmoid(jnp.dot(cur, tgw_ref[...], preferred_element_type=F32)
                          + tgb_ref[...])
    o_ref[...] = gate * cur + (1.0 - gate) * h


def _tc_final(h, s_p, t_p, d_p, w_n, l_w, tg_w, tg_b, block=1024):
    n_acc, d = h.shape
    grid = n_acc // block
    return pl.pallas_call(
        _tc_final_body,
        grid=(grid,),
        in_specs=[
            pl.BlockSpec((block, d), lambda i: (i, 0)),
            pl.BlockSpec((NC, block, d), lambda i: (0, i, 0)),
            pl.BlockSpec((NC, block, d), lambda i: (0, i, 0)),
            pl.BlockSpec((NW, block, 1), lambda i: (0, i, 0)),
            pl.BlockSpec((d, d), lambda i: (0, 0)),
            pl.BlockSpec((d, d), lambda i: (0, 0)),
            pl.BlockSpec((d, d), lambda i: (0, 0)),
            pl.BlockSpec((1, d), lambda i: (0, 0)),
        ],
        out_specs=pl.BlockSpec((block, d), lambda i: (i, 0)),
        out_shape=jax.ShapeDtypeStruct((n_acc, d), F32),
    )(h, s_p, t_p, d_p, w_n, l_w, tg_w, tg_b)


def kernel(edge_index, edge_type, dynamic_emb, emb_rel, W_ih, W_hh, b_ih, b_hh,
           w_neighbor, loop_weight, time_gate_weight, time_gate_bias):
    n, d = dynamic_emb.shape
    r = emb_rel.shape[0]
    e = edge_type.shape[0]

    ep = ((e + NW * CHUNK * 8 - 1) // (NW * CHUNK * 8)) * (NW * CHUNK * 8)
    pad = ep - e
    n_acc = ((n + NS * CHUNK) // (NS * CHUNK)) * (NS * CHUNK)
    r_acc = ((r + NS - 1) // NS + 1) * NS
    r_acc = ((r_acc + NS * 8 - 1) // (NS * 8)) * (NS * 8)

    src = edge_index[0]
    dst = edge_index[1]
    srci = jnp.concatenate([src, jnp.zeros((pad,), jnp.int32)]).reshape(-1, CHUNK)
    dsti = jnp.concatenate([dst, jnp.full((pad,), n, jnp.int32)]).reshape(-1, CHUNK)
    eti = jnp.concatenate([edge_type, jnp.full((pad,), r, jnp.int32)]).reshape(-1, CHUNK)

    de_p = jnp.zeros((n_acc, d), F32).at[:n].set(dynamic_emb)
    h_pad = _tc_norm(de_p)

    s_p, rel_p = _sc_pass1(h_pad, srci, dsti, eti, n_acc, r_acc)
    hist_flat = _sc_counts(dsti.reshape(-1), eti.reshape(-1), n_acc, r_acc)
    hist_flat = hist_flat.reshape(NW, n_acc + r_acc)
    deg_p = hist_flat[:, :n_acc].reshape(NW, n_acc, 1)
    cnt_p = hist_flat[:, n_acc:].reshape(NW, r_acc, 1)

    emb_p = jnp.zeros((r_acc, d), F32).at[:r].set(emb_rel)
    h0 = _tc_gru(rel_p, cnt_p, emb_p, W_ih, W_hh,
                 b_ih.reshape(1, -1), b_hh.reshape(1, -1))

    t_p = _sc_pass2(h0, dsti, eti, n_acc)

    h_new = _tc_final(h_pad, s_p, t_p, deg_p, w_neighbor, loop_weight,
                      time_gate_weight, time_gate_bias.reshape(1, -1))
    return h_new[:n]


# speedup vs baseline: 4.4977x; 1.0202x over previous
import functools

import jax
import jax.numpy as jnp
from jax import lax
from jax.experimental import pallas as pl
from jax.experimental.pallas import tpu as pltpu
from jax.experimental.pallas import tpu_sc as plsc

F32 = jnp.float32

NC = 2
NS = 16
NW = NC * NS
CHUNK = 64


def _l2norm(x):
    n = jnp.sqrt(jnp.sum(x * x, axis=-1, keepdims=True))
    return x / jnp.clip(n, 1e-12, None)


def _tc_norm_body(x_ref, o_ref):
    o_ref[...] = _l2norm(x_ref[...])


def _tc_norm(x, block=1024):
    m, d = x.shape
    grid = m // block
    return pl.pallas_call(
        _tc_norm_body,
        grid=(grid,),
        in_specs=[pl.BlockSpec((block, d), lambda i: (i, 0))],
        out_specs=pl.BlockSpec((block, d), lambda i: (i, 0)),
        out_shape=jax.ShapeDtypeStruct((m, d), F32),
    )(x)


GROUP = 8


def _sc1_body(n_acc, r_acc, ch,
              h_hbm, srci_hbm, dsti_hbm, eti_hbm,
              out_s, out_r,
              acc_s, acc_r,
              srci, dsti, eti, srci_b, dsti_b, eti_b,
              rows, rows2, rows3, sem, sem2, sem3,
              ssem, ssem2, ssem3, isem):
    cid = lax.axis_index("c")
    sid = lax.axis_index("s")
    wid = cid * NS + sid
    d = rows.shape[1]

    def fill_row(i, _):
        for j in range(d // 16):
            rows[i, pl.ds(j * 16, 16)] = jnp.zeros((16,), F32)
        return 0
    lax.fori_loop(0, CHUNK, fill_row, 0)

    n_per = n_acc // NS
    r_per = r_acc // NS
    for k in range(n_per // CHUNK):
        base = sid * n_per + k * CHUNK
        pltpu.sync_copy(rows, acc_s.at[pl.ds(base, CHUNK)])
    pltpu.sync_copy(rows.at[pl.ds(0, r_per)], acc_r.at[pl.ds(sid * r_per, r_per)])
    plsc.subcore_barrier()

    def chunks(si, di, ei):
        bufs = (rows, rows2, rows3)
        gsems = (sem, sem2, sem3)
        ssems = (ssem, ssem2, ssem3)
        gd = {}
        sd = {}
        gd[0] = pltpu.async_copy(h_hbm.at[si.at[0]], bufs[0], gsems[0])
        for c in range(GROUP):
            b = c % 3
            if c + 1 < GROUP:
                bn = (c + 1) % 3
                if c - 2 >= 0:
                    sd[c - 2][0].wait()
                    sd[c - 2][1].wait()
                gd[c + 1] = pltpu.async_copy(
                    h_hbm.at[si.at[c + 1]], bufs[bn], gsems[bn])
            gd[c].wait()
            sd[c] = (
                pltpu.async_copy(bufs[b], acc_s.at[di.at[c]], ssems[b], add=True),
                pltpu.async_copy(bufs[b], acc_r.at[ei.at[c]], ssems[b], add=True),
            )
        for c in (GROUP - 3, GROUP - 2, GROUP - 1):
            sd[c][0].wait()
            sd[c][1].wait()

    def pair(p, _):
        base0 = pl.multiple_of(wid * ch + (2 * p) * GROUP, GROUP)
        base1 = pl.multiple_of(wid * ch + (2 * p + 1) * GROUP, GROUP)
        pltpu.sync_copy(srci_hbm.at[pl.ds(base0, GROUP)], srci)
        pltpu.sync_copy(dsti_hbm.at[pl.ds(base0, GROUP)], dsti)
        pltpu.sync_copy(eti_hbm.at[pl.ds(base0, GROUP)], eti)
        pf = (
            pltpu.async_copy(srci_hbm.at[pl.ds(base1, GROUP)], srci_b, isem),
            pltpu.async_copy(dsti_hbm.at[pl.ds(base1, GROUP)], dsti_b, isem),
            pltpu.async_copy(eti_hbm.at[pl.ds(base1, GROUP)], eti_b, isem),
        )
        chunks(srci, dsti, eti)
        for x in pf:
            x.wait()
        chunks(srci_b, dsti_b, eti_b)
        return 0
    lax.fori_loop(0, ch // GROUP // 2, pair, 0)
    plsc.subcore_barrier()

    for k in range(n_per // CHUNK):
        base = sid * n_per + k * CHUNK
        pltpu.sync_copy(acc_s.at[pl.ds(base, CHUNK)], out_s.at[cid, pl.ds(base, CHUNK)])
    pltpu.sync_copy(acc_r.at[pl.ds(sid * r_per, r_per)], out_r.at[cid, pl.ds(sid * r_per, r_per)])


def _sc_pass1(h, srci, dsti, eti, n_acc, r_acc):
    d = h.shape[1]
    ch = srci.shape[0] // NW
    mesh = plsc.VectorSubcoreMesh(core_axis_name="c", subcore_axis_name="s",
                                  num_cores=NC, num_subcores=NS)
    kern = pl.kernel(
        functools.partial(_sc1_body, n_acc, r_acc, ch),
        out_type=(
            jax.ShapeDtypeStruct((NC, n_acc, d), F32),
            jax.ShapeDtypeStruct((NC, r_acc, d), F32),
        ),
        mesh=mesh,
        scratch_types=[
            pltpu.VMEM_SHARED((n_acc, d), F32),
            pltpu.VMEM_SHARED((r_acc, d), F32),
            pltpu.VMEM((GROUP, CHUNK), jnp.int32),
            pltpu.VMEM((GROUP, CHUNK), jnp.int32),
            pltpu.VMEM((GROUP, CHUNK), jnp.int32),
            pltpu.VMEM((GROUP, CHUNK), jnp.int32),
            pltpu.VMEM((GROUP, CHUNK), jnp.int32),
            pltpu.VMEM((GROUP, CHUNK), jnp.int32),
            pltpu.VMEM((CHUNK, d), F32),
            pltpu.VMEM((CHUNK, d), F32),
            pltpu.VMEM((CHUNK, d), F32),
            pltpu.SemaphoreType.DMA,
            pltpu.SemaphoreType.DMA,
            pltpu.SemaphoreType.DMA,
            pltpu.SemaphoreType.DMA,
            pltpu.SemaphoreType.DMA,
            pltpu.SemaphoreType.DMA,
            pltpu.SemaphoreType.DMA,
        ],
    )
    return kern(h, srci, dsti, eti)


CGROUP = 512


def _scc_body(n_acc, hsize, epw,
              dsti_hbm, eti_hbm, out_h,
              idx_d, idx_e, hist):
    cid = lax.axis_index("c")
    sid = lax.axis_index("s")
    wid = cid * NS + sid

    def fill(i, _):
        hist[pl.ds(i * 16, 16)] = jnp.zeros((16,), F32)
        return 0
    lax.fori_loop(0, hsize // 16, fill, 0)

    def group(g, _):
        base = pl.multiple_of(wid * epw + g * CGROUP, 8)
        pltpu.sync_copy(dsti_hbm.at[pl.ds(base, CGROUP)], idx_d)
        pltpu.sync_copy(eti_hbm.at[pl.ds(base, CGROUP)], idx_e)

        def step(k, _):
            dv = idx_d[pl.ds(k * 16, 16)]
            cnts, last = plsc.scan_count(dv)
            plsc.addupdate_scatter(hist, [dv], cnts.astype(F32), mask=last)
            ev = idx_e[pl.ds(k * 16, 16)] + n_acc
            cnts2, last2 = plsc.scan_count(ev)
            plsc.addupdate_scatter(hist, [ev], cnts2.astype(F32), mask=last2)
            return 0
        lax.fori_loop(0, CGROUP // 16, step, 0)
        return 0
    lax.fori_loop(0, epw // CGROUP, group, 0)

    pltpu.sync_copy(hist, out_h.at[pl.ds(wid * hsize, hsize)])


def _sc_counts(dsti_flat, eti_flat, n_acc, r_acc):
    ep = dsti_flat.shape[0]
    epw = ep // NW
    hsize = n_acc + r_acc
    assert epw % CGROUP == 0 and hsize % 16 == 0 and (hsize % 8 == 0)
    mesh = plsc.VectorSubcoreMesh(core_axis_name="c", subcore_axis_name="s",
                                  num_cores=NC, num_subcores=NS)
    kern = pl.kernel(
        functools.partial(_scc_body, n_acc, hsize, epw),
        out_type=jax.ShapeDtypeStruct((NW * hsize,), F32),
        mesh=mesh,
        scratch_types=[
            pltpu.VMEM((CGROUP,), jnp.int32),
            pltpu.VMEM((CGROUP,), jnp.int32),
            pltpu.VMEM((hsize,), F32),
        ],
        compiler_params=pltpu.CompilerParams(needs_layout_passes=False),
    )
    return kern(dsti_flat, eti_flat)


def _sc2_body(n_acc, ch,
              h0_hbm, dsti_hbm, eti_hbm, out_t,
              acc_t, h0_sp, dsti, eti, dsti_b, eti_b,
              rows, rows2, rows3, sem, sem2, sem3,
              ssem, ssem2, ssem3, isem):
    cid = lax.axis_index("c")
    sid = lax.axis_index("s")
    wid = cid * NS + sid
    d = rows.shape[1]

    def fill_row(i, _):
        for j in range(d // 16):
            rows[i, pl.ds(j * 16, 16)] = jnp.zeros((16,), F32)
        return 0
    lax.fori_loop(0, CHUNK, fill_row, 0)

    n_per = n_acc // NS
    for k in range(n_per // CHUNK):
        base = sid * n_per + k * CHUNK
        pltpu.sync_copy(rows, acc_t.at[pl.ds(base, CHUNK)])
    @pl.when(sid == 0)
    def _():
        pltpu.sync_copy(h0_hbm, h0_sp)
    plsc.subcore_barrier()

    def chunks(di, ei):
        bufs = (rows, rows2, rows3)
        gsems = (sem, sem2, sem3)
        ssems = (ssem, ssem2, ssem3)
        gd = {}
        sd = {}
        gd[0] = pltpu.async_copy(h0_sp.at[ei.at[0]], bufs[0], gsems[0])
        for c in range(GROUP):
            b = c % 3
            if c + 1 < GROUP:
                bn = (c + 1) % 3
                if c - 2 >= 0:
                    sd[c - 2].wait()
                gd[c + 1] = pltpu.async_copy(
                    h0_sp.at[ei.at[c + 1]], bufs[bn], gsems[bn])
            gd[c].wait()
            sd[c] = pltpu.async_copy(bufs[b], acc_t.at[di.at[c]], ssems[b], add=True)
        for c in (GROUP - 3, GROUP - 2, GROUP - 1):
            sd[c].wait()

    def pair(p, _):
        base0 = pl.multiple_of(wid * ch + (2 * p) * GROUP, GROUP)
        base1 = pl.multiple_of(wid * ch + (2 * p + 1) * GROUP, GROUP)
        pltpu.sync_copy(dsti_hbm.at[pl.ds(base0, GROUP)], dsti)
        pltpu.sync_copy(eti_hbm.at[pl.ds(base0, GROUP)], eti)
        pf = (
            pltpu.async_copy(dsti_hbm.at[pl.ds(base1, GROUP)], dsti_b, isem),
            pltpu.async_copy(eti_hbm.at[pl.ds(base1, GROUP)], eti_b, isem),
        )
        chunks(dsti, eti)
        for x in pf:
            x.wait()
        chunks(dsti_b, eti_b)
        return 0
    lax.fori_loop(0, ch // GROUP // 2, pair, 0)
    plsc.subcore_barrier()

    for k in range(n_per // CHUNK):
        base = sid * n_per + k * CHUNK
        pltpu.sync_copy(acc_t.at[pl.ds(base, CHUNK)], out_t.at[cid, pl.ds(base, CHUNK)])


def _sc_pass2(h0, dsti, eti, n_acc):
    r_acc, d = h0.shape
    ch = dsti.shape[0] // NW
    mesh = plsc.VectorSubcoreMesh(core_axis_name="c", subcore_axis_name="s",
                                  num_cores=NC, num_subcores=NS)
    kern = pl.kernel(
        functools.partial(_sc2_body, n_acc, ch),
        out_type=jax.ShapeDtypeStruct((NC, n_acc, d), F32),
        mesh=mesh,
        scratch_types=[
            pltpu.VMEM_SHARED((n_acc, d), F32),
            pltpu.VMEM_SHARED((r_acc, d), F32),
            pltpu.VMEM((GROUP, CHUNK), jnp.int32),
            pltpu.VMEM((GROUP, CHUNK), jnp.int32),
            pltpu.VMEM((GROUP, CHUNK), jnp.int32),
            pltpu.VMEM((GROUP, CHUNK), jnp.int32),
            pltpu.VMEM((CHUNK, d), F32),
            pltpu.VMEM((CHUNK, d), F32),
            pltpu.VMEM((CHUNK, d), F32),
            pltpu.SemaphoreType.DMA,
            pltpu.SemaphoreType.DMA,
            pltpu.SemaphoreType.DMA,
            pltpu.SemaphoreType.DMA,
            pltpu.SemaphoreType.DMA,
            pltpu.SemaphoreType.DMA,
            pltpu.SemaphoreType.DMA,
        ],
    )
    return kern(h0, dsti, eti)


def _tc_gru_body(relp_ref, cntp_ref, emb_ref, wih_ref, whh_ref, bih_ref, bhh_ref, o_ref):
    d = emb_ref.shape[1]
    rel_sum = relp_ref[0] + relp_ref[1]
    cnt = jnp.sum(cntp_ref[...], axis=0)
    rel_mean = rel_sum / jnp.maximum(cnt, 1.0)
    emb = emb_ref[...]
    x = jnp.concatenate([emb, rel_mean], axis=1)
    gi = lax.dot_general(x, wih_ref[...], (((1,), (1,)), ((), ())),
                         preferred_element_type=F32) + bih_ref[...]
    gh = lax.dot_general(emb, whh_ref[...], (((1,), (1,)), ((), ())),
                         preferred_element_type=F32) + bhh_ref[...]
    i_r, i_z, i_n = gi[:, :d], gi[:, d:2 * d], gi[:, 2 * d:]
    h_r, h_z, h_n = gh[:, :d], gh[:, d:2 * d], gh[:, 2 * d:]
    r = jax.nn.sigmoid(i_r + h_r)
    z = jax.nn.sigmoid(i_z + h_z)
    n = jnp.tanh(i_n + r * h_n)
    h0 = (1.0 - z) * n + z * emb
    o_ref[...] = _l2norm(h0)


def _tc_gru(rel_p, cnt_p, emb_p, w_ih, w_hh, b_ih, b_hh):
    r_acc, d = emb_p.shape
    return pl.pallas_call(
        _tc_gru_body,
        out_shape=jax.ShapeDtypeStruct((r_acc, d), F32),
    )(rel_p, cnt_p, emb_p, w_ih, w_hh, b_ih, b_hh)


def _tc_final_body(h_ref, s_ref, t_ref, d_ref, wn_ref, lw_ref, tgw_ref, tgb_ref, o_ref):
    u = (s_ref[0] - t_ref[0]) + (s_ref[1] - t_ref[1])
    deg = jnp.sum(d_ref[...], axis=0)
    agg = jnp.dot(u, wn_ref[...], preferred_element_type=F32) / jnp.maximum(deg, 1.0)
    h = h_ref[...]
    cur = agg + jnp.dot(h, lw_ref[...], preferred_element_type=F32)
    slope = (1.0 / 8.0 + 1.0 / 3.0) / 2.0
    cur = jnp.where(cur >= 0, cur, slope * cur)
    cur = _l2norm(cur)
    gate = jax.nn.sigmoid(jnp.dot(cur, tgw_ref[...], preferred_element_type=F32)
                          + tgb_ref[...])
    o_ref[...] = gate * cur + (1.0 - gate) * h


def _tc_final(h, s_p, t_p, d_p, w_n, l_w, tg_w, tg_b, block=1024):
    n_acc, d = h.shape
    grid = n_acc // block
    return pl.pallas_call(
        _tc_final_body,
        grid=(grid,),
        in_specs=[
            pl.BlockSpec((block, d), lambda i: (i, 0)),
            pl.BlockSpec((NC, block, d), lambda i: (0, i, 0)),
            pl.BlockSpec((NC, block, d), lambda i: (0, i, 0)),
            pl.BlockSpec((NW, block, 1), lambda i: (0, i, 0)),
            pl.BlockSpec((d, d), lambda i: (0, 0)),
            pl.BlockSpec((d, d), lambda i: (0, 0)),
            pl.BlockSpec((d, d), lambda i: (0, 0)),
            pl.BlockSpec((1, d), lambda i: (0, 0)),
        ],
        out_specs=pl.BlockSpec((block, d), lambda i: (i, 0)),
        out_shape=jax.ShapeDtypeStruct((n_acc, d), F32),
    )(h, s_p, t_p, d_p, w_n, l_w, tg_w, tg_b)


def kernel(edge_index, edge_type, dynamic_emb, emb_rel, W_ih, W_hh, b_ih, b_hh,
           w_neighbor, loop_weight, time_gate_weight, time_gate_bias):
    n, d = dynamic_emb.shape
    r = emb_rel.shape[0]
    e = edge_type.shape[0]

    ep = ((e + NW * CHUNK * 16 - 1) // (NW * CHUNK * 16)) * (NW * CHUNK * 16)
    pad = ep - e
    n_acc = ((n + NS * CHUNK) // (NS * CHUNK)) * (NS * CHUNK)
    r_acc = ((r + NS - 1) // NS + 1) * NS
    r_acc = ((r_acc + NS * 8 - 1) // (NS * 8)) * (NS * 8)

    src = edge_index[0]
    dst = edge_index[1]
    srci = jnp.concatenate([src, jnp.zeros((pad,), jnp.int32)]).reshape(-1, CHUNK)
    dsti = jnp.concatenate([dst, jnp.full((pad,), n, jnp.int32)]).reshape(-1, CHUNK)
    eti = jnp.concatenate([edge_type, jnp.full((pad,), r, jnp.int32)]).reshape(-1, CHUNK)

    de_p = jnp.zeros((n_acc, d), F32).at[:n].set(dynamic_emb)
    h_pad = _tc_norm(de_p)

    s_p, rel_p = _sc_pass1(h_pad, srci, dsti, eti, n_acc, r_acc)
    hist_flat = _sc_counts(dsti.reshape(-1), eti.reshape(-1), n_acc, r_acc)
    hist_flat = hist_flat.reshape(NW, n_acc + r_acc)
    deg_p = hist_flat[:, :n_acc].reshape(NW, n_acc, 1)
    cnt_p = hist_flat[:, n_acc:].reshape(NW, r_acc, 1)

    emb_p = jnp.zeros((r_acc, d), F32).at[:r].set(emb_rel)
    h0 = _tc_gru(rel_p, cnt_p, emb_p, W_ih, W_hh,
                 b_ih.reshape(1, -1), b_hh.reshape(1, -1))

    t_p = _sc_pass2(h0, dsti, eti, n_acc)

    h_new = _tc_final(h_pad, s_p, t_p, deg_p, w_neighbor, loop_weight,
                      time_gate_weight, time_gate_bias.reshape(1, -1))
    return h_new[:n]


# speedup vs baseline: 4.8499x; 1.0783x over previous
import functools

import jax
import jax.numpy as jnp
from jax import lax
from jax.experimental import pallas as pl
from jax.experimental.pallas import tpu as pltpu
from jax.experimental.pallas import tpu_sc as plsc

F32 = jnp.float32

NC = 2
NS = 16
NW = NC * NS
CHUNK = 64


def _l2norm(x):
    n = jnp.sqrt(jnp.sum(x * x, axis=-1, keepdims=True))
    return x / jnp.clip(n, 1e-12, None)


def _tc_norm_body(x_ref, o_ref):
    o_ref[...] = _l2norm(x_ref[...])


def _tc_norm(x, block=1024):
    m, d = x.shape
    grid = m // block
    return pl.pallas_call(
        _tc_norm_body,
        grid=(grid,),
        in_specs=[pl.BlockSpec((block, d), lambda i: (i, 0))],
        out_specs=pl.BlockSpec((block, d), lambda i: (i, 0)),
        out_shape=jax.ShapeDtypeStruct((m, d), F32),
    )(x)


GROUP = 8


def _sc1_body(n_acc, r_acc, ch,
              h_hbm, srci_hbm, dsti_hbm, eti_hbm,
              out_s, out_r,
              acc_s, acc_r,
              srci, dsti, eti, srci_b, dsti_b, eti_b,
              rows, rows2, rows3, sem, sem2, sem3,
              ssem, ssem2, ssem3, isem):
    cid = lax.axis_index("c")
    sid = lax.axis_index("s")
    wid = cid * NS + sid
    d = rows.shape[1]

    def fill_row(i, _):
        for j in range(d // 16):
            rows[i, pl.ds(j * 16, 16)] = jnp.zeros((16,), F32)
        return 0
    lax.fori_loop(0, CHUNK, fill_row, 0)

    n_per = n_acc // NS
    r_per = r_acc // NS
    for k in range(n_per // CHUNK):
        base = sid * n_per + k * CHUNK
        pltpu.sync_copy(rows, acc_s.at[pl.ds(base, CHUNK)])
    pltpu.sync_copy(rows.at[pl.ds(0, r_per)], acc_r.at[pl.ds(sid * r_per, r_per)])
    plsc.subcore_barrier()

    def chunks(si, di, ei):
        bufs = (rows, rows2, rows3)
        gsems = (sem, sem2, sem3)
        ssems = (ssem, ssem2, ssem3)
        gd = {}
        sd = {}
        gd[0] = pltpu.async_copy(h_hbm.at[si.at[0]], bufs[0], gsems[0])
        for c in range(GROUP):
            b = c % 3
            if c + 1 < GROUP:
                bn = (c + 1) % 3
                if c - 2 >= 0:
                    sd[c - 2][0].wait()
                    sd[c - 2][1].wait()
                gd[c + 1] = pltpu.async_copy(
                    h_hbm.at[si.at[c + 1]], bufs[bn], gsems[bn])
            gd[c].wait()
            sd[c] = (
                pltpu.async_copy(bufs[b], acc_s.at[di.at[c]], ssems[b], add=True),
                pltpu.async_copy(bufs[b], acc_r.at[ei.at[c]], ssems[b], add=True),
            )
        for c in (GROUP - 3, GROUP - 2, GROUP - 1):
            sd[c][0].wait()
            sd[c][1].wait()

    ch0 = (ch * 2 * 7 // 10) // 16 * 16
    ch1 = ch * 2 - ch0

    def run(nch, tile_base):
        def pair(p, _):
            base0 = pl.multiple_of(tile_base + (2 * p) * GROUP, GROUP)
            base1 = pl.multiple_of(tile_base + (2 * p + 1) * GROUP, GROUP)
            pltpu.sync_copy(srci_hbm.at[pl.ds(base0, GROUP)], srci)
            pltpu.sync_copy(dsti_hbm.at[pl.ds(base0, GROUP)], dsti)
            pltpu.sync_copy(eti_hbm.at[pl.ds(base0, GROUP)], eti)
            pf = (
                pltpu.async_copy(srci_hbm.at[pl.ds(base1, GROUP)], srci_b, isem),
                pltpu.async_copy(dsti_hbm.at[pl.ds(base1, GROUP)], dsti_b, isem),
                pltpu.async_copy(eti_hbm.at[pl.ds(base1, GROUP)], eti_b, isem),
            )
            chunks(srci, dsti, eti)
            for x in pf:
                x.wait()
            chunks(srci_b, dsti_b, eti_b)
            return 0
        lax.fori_loop(0, nch // GROUP // 2, pair, 0)

    @pl.when(cid == 0)
    def _():
        run(ch0, sid * ch0)

    @pl.when(cid == 1)
    def _():
        run(ch1, NS * ch0 + sid * ch1)
    plsc.subcore_barrier()

    for k in range(n_per // CHUNK):
        base = sid * n_per + k * CHUNK
        pltpu.sync_copy(acc_s.at[pl.ds(base, CHUNK)], out_s.at[cid, pl.ds(base, CHUNK)])
    pltpu.sync_copy(acc_r.at[pl.ds(sid * r_per, r_per)], out_r.at[cid, pl.ds(sid * r_per, r_per)])


def _sc_pass1(h, srci, dsti, eti, n_acc, r_acc):
    d = h.shape[1]
    ch = srci.shape[0] // NW
    mesh = plsc.VectorSubcoreMesh(core_axis_name="c", subcore_axis_name="s",
                                  num_cores=NC, num_subcores=NS)
    kern = pl.kernel(
        functools.partial(_sc1_body, n_acc, r_acc, ch),
        out_type=(
            jax.ShapeDtypeStruct((NC, n_acc, d), F32),
            jax.ShapeDtypeStruct((NC, r_acc, d), F32),
        ),
        mesh=mesh,
        scratch_types=[
            pltpu.VMEM_SHARED((n_acc, d), F32),
            pltpu.VMEM_SHARED((r_acc, d), F32),
            pltpu.VMEM((GROUP, CHUNK), jnp.int32),
            pltpu.VMEM((GROUP, CHUNK), jnp.int32),
            pltpu.VMEM((GROUP, CHUNK), jnp.int32),
            pltpu.VMEM((GROUP, CHUNK), jnp.int32),
            pltpu.VMEM((GROUP, CHUNK), jnp.int32),
            pltpu.VMEM((GROUP, CHUNK), jnp.int32),
            pltpu.VMEM((CHUNK, d), F32),
            pltpu.VMEM((CHUNK, d), F32),
            pltpu.VMEM((CHUNK, d), F32),
            pltpu.SemaphoreType.DMA,
            pltpu.SemaphoreType.DMA,
            pltpu.SemaphoreType.DMA,
            pltpu.SemaphoreType.DMA,
            pltpu.SemaphoreType.DMA,
            pltpu.SemaphoreType.DMA,
            pltpu.SemaphoreType.DMA,
        ],
    )
    return kern(h, srci, dsti, eti)


CGROUP = 512


def _scc_body(n_acc, hsize, epw,
              dsti_hbm, eti_hbm, out_h,
              idx_d, idx_e, hist):
    cid = lax.axis_index("c")
    sid = lax.axis_index("s")
    wid = cid * NS + sid

    def fill(i, _):
        hist[pl.ds(i * 16, 16)] = jnp.zeros((16,), F32)
        return 0
    lax.fori_loop(0, hsize // 16, fill, 0)

    def group(g, _):
        base = pl.multiple_of(wid * epw + g * CGROUP, 8)
        pltpu.sync_copy(dsti_hbm.at[pl.ds(base, CGROUP)], idx_d)
        pltpu.sync_copy(eti_hbm.at[pl.ds(base, CGROUP)], idx_e)

        def step(k, _):
            dv = idx_d[pl.ds(k * 16, 16)]
            cnts, last = plsc.scan_count(dv)
            plsc.addupdate_scatter(hist, [dv], cnts.astype(F32), mask=last)
            ev = idx_e[pl.ds(k * 16, 16)] + n_acc
            cnts2, last2 = plsc.scan_count(ev)
            plsc.addupdate_scatter(hist, [ev], cnts2.astype(F32), mask=last2)
            return 0
        lax.fori_loop(0, CGROUP // 16, step, 0)
        return 0
    lax.fori_loop(0, epw // CGROUP, group, 0)

    pltpu.sync_copy(hist, out_h.at[pl.ds(wid * hsize, hsize)])


def _sc_counts(dsti_flat, eti_flat, n_acc, r_acc):
    ep = dsti_flat.shape[0]
    epw = ep // NW
    hsize = n_acc + r_acc
    assert epw % CGROUP == 0 and hsize % 16 == 0 and (hsize % 8 == 0)
    mesh = plsc.VectorSubcoreMesh(core_axis_name="c", subcore_axis_name="s",
                                  num_cores=NC, num_subcores=NS)
    kern = pl.kernel(
        functools.partial(_scc_body, n_acc, hsize, epw),
        out_type=jax.ShapeDtypeStruct((NW * hsize,), F32),
        mesh=mesh,
        scratch_types=[
            pltpu.VMEM((CGROUP,), jnp.int32),
            pltpu.VMEM((CGROUP,), jnp.int32),
            pltpu.VMEM((hsize,), F32),
        ],
        compiler_params=pltpu.CompilerParams(needs_layout_passes=False),
    )
    return kern(dsti_flat, eti_flat)


def _sc2_body(n_acc, ch,
              h0_hbm, dsti_hbm, eti_hbm, out_t,
              acc_t, h0_sp, dsti, eti, dsti_b, eti_b,
              rows, rows2, rows3, sem, sem2, sem3,
              ssem, ssem2, ssem3, isem):
    cid = lax.axis_index("c")
    sid = lax.axis_index("s")
    wid = cid * NS + sid
    d = rows.shape[1]

    def fill_row(i, _):
        for j in range(d // 16):
            rows[i, pl.ds(j * 16, 16)] = jnp.zeros((16,), F32)
        return 0
    lax.fori_loop(0, CHUNK, fill_row, 0)

    n_per = n_acc // NS
    for k in range(n_per // CHUNK):
        base = sid * n_per + k * CHUNK
        pltpu.sync_copy(rows, acc_t.at[pl.ds(base, CHUNK)])
    @pl.when(sid == 0)
    def _():
        pltpu.sync_copy(h0_hbm, h0_sp)
    plsc.subcore_barrier()

    def chunks(di, ei):
        bufs = (rows, rows2, rows3)
        gsems = (sem, sem2, sem3)
        ssems = (ssem, ssem2, ssem3)
        gd = {}
        sd = {}
        gd[0] = pltpu.async_copy(h0_sp.at[ei.at[0]], bufs[0], gsems[0])
        for c in range(GROUP):
            b = c % 3
            if c + 1 < GROUP:
                bn = (c + 1) % 3
                if c - 2 >= 0:
                    sd[c - 2].wait()
                gd[c + 1] = pltpu.async_copy(
                    h0_sp.at[ei.at[c + 1]], bufs[bn], gsems[bn])
            gd[c].wait()
            sd[c] = pltpu.async_copy(bufs[b], acc_t.at[di.at[c]], ssems[b], add=True)
        for c in (GROUP - 3, GROUP - 2, GROUP - 1):
            sd[c].wait()

    def pair(p, _):
        base0 = pl.multiple_of(wid * ch + (2 * p) * GROUP, GROUP)
        base1 = pl.multiple_of(wid * ch + (2 * p + 1) * GROUP, GROUP)
        pltpu.sync_copy(dsti_hbm.at[pl.ds(base0, GROUP)], dsti)
        pltpu.sync_copy(eti_hbm.at[pl.ds(base0, GROUP)], eti)
        pf = (
            pltpu.async_copy(dsti_hbm.at[pl.ds(base1, GROUP)], dsti_b, isem),
            pltpu.async_copy(eti_hbm.at[pl.ds(base1, GROUP)], eti_b, isem),
        )
        chunks(dsti, eti)
        for x in pf:
            x.wait()
        chunks(dsti_b, eti_b)
        return 0
    lax.fori_loop(0, ch // GROUP // 2, pair, 0)
    plsc.subcore_barrier()

    for k in range(n_per // CHUNK):
        base = sid * n_per + k * CHUNK
        pltpu.sync_copy(acc_t.at[pl.ds(base, CHUNK)], out_t.at[cid, pl.ds(base, CHUNK)])


def _sc_pass2(h0, dsti, eti, n_acc):
    r_acc, d = h0.shape
    ch = dsti.shape[0] // NW
    mesh = plsc.VectorSubcoreMesh(core_axis_name="c", subcore_axis_name="s",
                                  num_cores=NC, num_subcores=NS)
    kern = pl.kernel(
        functools.partial(_sc2_body, n_acc, ch),
        out_type=jax.ShapeDtypeStruct((NC, n_acc, d), F32),
        mesh=mesh,
        scratch_types=[
            pltpu.VMEM_SHARED((n_acc, d), F32),
            pltpu.VMEM_SHARED((r_acc, d), F32),
            pltpu.VMEM((GROUP, CHUNK), jnp.int32),
            pltpu.VMEM((GROUP, CHUNK), jnp.int32),
            pltpu.VMEM((GROUP, CHUNK), jnp.int32),
            pltpu.VMEM((GROUP, CHUNK), jnp.int32),
            pltpu.VMEM((CHUNK, d), F32),
            pltpu.VMEM((CHUNK, d), F32),
            pltpu.VMEM((CHUNK, d), F32),
            pltpu.SemaphoreType.DMA,
            pltpu.SemaphoreType.DMA,
            pltpu.SemaphoreType.DMA,
            pltpu.SemaphoreType.DMA,
            pltpu.SemaphoreType.DMA,
            pltpu.SemaphoreType.DMA,
            pltpu.SemaphoreType.DMA,
        ],
    )
    return kern(h0, dsti, eti)


def _tc_gru_body(relp_ref, cntp_ref, emb_ref, wih_ref, whh_ref, bih_ref, bhh_ref, o_ref):
    d = emb_ref.shape[1]
    rel_sum = relp_ref[0] + relp_ref[1]
    cnt = jnp.sum(cntp_ref[...], axis=0)
    rel_mean = rel_sum / jnp.maximum(cnt, 1.0)
    emb = emb_ref[...]
    x = jnp.concatenate([emb, rel_mean], axis=1)
    gi = lax.dot_general(x, wih_ref[...], (((1,), (1,)), ((), ())),
                         preferred_element_type=F32) + bih_ref[...]
    gh = lax.dot_general(emb, whh_ref[...], (((1,), (1,)), ((), ())),
                         preferred_element_type=F32) + bhh_ref[...]
    i_r, i_z, i_n = gi[:, :d], gi[:, d:2 * d], gi[:, 2 * d:]
    h_r, h_z, h_n = gh[:, :d], gh[:, d:2 * d], gh[:, 2 * d:]
    r = jax.nn.sigmoid(i_r + h_r)
    z = jax.nn.sigmoid(i_z + h_z)
    n = jnp.tanh(i_n + r * h_n)
    h0 = (1.0 - z) * n + z * emb
    o_ref[...] = _l2norm(h0)


def _tc_gru(rel_p, cnt_p, emb_p, w_ih, w_hh, b_ih, b_hh):
    r_acc, d = emb_p.shape
    return pl.pallas_call(
        _tc_gru_body,
        out_shape=jax.ShapeDtypeStruct((r_acc, d), F32),
    )(rel_p, cnt_p, emb_p, w_ih, w_hh, b_ih, b_hh)


def _tc_final_body(h_ref, s_ref, t_ref, d_ref, wn_ref, lw_ref, tgw_ref, tgb_ref, o_ref):
    u = (s_ref[0] - t_ref[0]) + (s_ref[1] - t_ref[1])
    deg = jnp.sum(d_ref[...], axis=0)
    agg = jnp.dot(u, wn_ref[...], preferred_element_type=F32) / jnp.maximum(deg, 1.0)
    h = h_ref[...]
    cur = agg + jnp.dot(h, lw_ref[...], preferred_element_type=F32)
    slope = (1.0 / 8.0 + 1.0 / 3.0) / 2.0
    cur = jnp.where(cur >= 0, cur, slope * cur)
    cur = _l2norm(cur)
    gate = jax.nn.sigmoid(jnp.dot(cur, tgw_ref[...], preferred_element_type=F32)
                          + tgb_ref[...])
    o_ref[...] = gate * cur + (1.0 - gate) * h


def _tc_final(h, s_p, t_p, d_p, w_n, l_w, tg_w, tg_b, block=1024):
    n_acc, d = h.shape
    grid = n_acc // block
    return pl.pallas_call(
        _tc_final_body,
        grid=(grid,),
        in_specs=[
            pl.BlockSpec((block, d), lambda i: (i, 0)),
            pl.BlockSpec((NC, block, d), lambda i: (0, i, 0)),
            pl.BlockSpec((NC, block, d), lambda i: (0, i, 0)),
            pl.BlockSpec((NW, block, 1), lambda i: (0, i, 0)),
            pl.BlockSpec((d, d), lambda i: (0, 0)),
            pl.BlockSpec((d, d), lambda i: (0, 0)),
            pl.BlockSpec((d, d), lambda i: (0, 0)),
            pl.BlockSpec((1, d), lambda i: (0, 0)),
        ],
        out_specs=pl.BlockSpec((block, d), lambda i: (i, 0)),
        out_shape=jax.ShapeDtypeStruct((n_acc, d), F32),
    )(h, s_p, t_p, d_p, w_n, l_w, tg_w, tg_b)


def kernel(edge_index, edge_type, dynamic_emb, emb_rel, W_ih, W_hh, b_ih, b_hh,
           w_neighbor, loop_weight, time_gate_weight, time_gate_bias):
    n, d = dynamic_emb.shape
    r = emb_rel.shape[0]
    e = edge_type.shape[0]

    ep = ((e + NW * CHUNK * 16 - 1) // (NW * CHUNK * 16)) * (NW * CHUNK * 16)
    pad = ep - e
    n_acc = ((n + NS * CHUNK) // (NS * CHUNK)) * (NS * CHUNK)
    r_acc = ((r + NS - 1) // NS + 1) * NS
    r_acc = ((r_acc + NS * 8 - 1) // (NS * 8)) * (NS * 8)

    src = edge_index[0]
    dst = edge_index[1]
    srci = jnp.concatenate([src, jnp.zeros((pad,), jnp.int32)]).reshape(-1, CHUNK)
    dsti = jnp.concatenate([dst, jnp.full((pad,), n, jnp.int32)]).reshape(-1, CHUNK)
    eti = jnp.concatenate([edge_type, jnp.full((pad,), r, jnp.int32)]).reshape(-1, CHUNK)

    de_p = jnp.zeros((n_acc, d), F32).at[:n].set(dynamic_emb)
    h_pad = _tc_norm(de_p)

    s_p, rel_p = _sc_pass1(h_pad, srci, dsti, eti, n_acc, r_acc)
    hist_flat = _sc_counts(dsti.reshape(-1), eti.reshape(-1), n_acc, r_acc)
    hist_flat = hist_flat.reshape(NW, n_acc + r_acc)
    deg_p = hist_flat[:, :n_acc].reshape(NW, n_acc, 1)
    cnt_p = hist_flat[:, n_acc:].reshape(NW, r_acc, 1)

    emb_p = jnp.zeros((r_acc, d), F32).at[:r].set(emb_rel)
    h0 = _tc_gru(rel_p, cnt_p, emb_p, W_ih, W_hh,
                 b_ih.reshape(1, -1), b_hh.reshape(1, -1))

    t_p = _sc_pass2(h0, dsti, eti, n_acc)

    h_new = _tc_final(h_pad, s_p, t_p, deg_p, w_neighbor, loop_weight,
                      time_gate_weight, time_gate_bias.reshape(1, -1))
    return h_new[:n]


# speedup vs baseline: 4.9239x; 1.0153x over previous
import functools

import jax
import jax.numpy as jnp
from jax import lax
from jax.experimental import pallas as pl
from jax.experimental.pallas import tpu as pltpu
from jax.experimental.pallas import tpu_sc as plsc

F32 = jnp.float32

NC = 2
NS = 16
NW = NC * NS
CHUNK = 64


def _l2norm(x):
    n = jnp.sqrt(jnp.sum(x * x, axis=-1, keepdims=True))
    return x / jnp.clip(n, 1e-12, None)


def _tc_norm_body(x_ref, o_ref):
    o_ref[...] = _l2norm(x_ref[...])


def _tc_norm(x, block=1024):
    m, d = x.shape
    grid = m // block
    return pl.pallas_call(
        _tc_norm_body,
        grid=(grid,),
        in_specs=[pl.BlockSpec((block, d), lambda i: (i, 0))],
        out_specs=pl.BlockSpec((block, d), lambda i: (i, 0)),
        out_shape=jax.ShapeDtypeStruct((m, d), F32),
    )(x)


GROUP = 8


def _sc1_body(n_acc, r_acc, ch,
              h_hbm, srci_hbm, dsti_hbm, eti_hbm,
              out_s, out_r,
              acc_s, acc_r,
              srci, dsti, eti, srci_b, dsti_b, eti_b,
              rows, rows2, rows3, sem, sem2, sem3,
              ssem, ssem2, ssem3, isem):
    cid = lax.axis_index("c")
    sid = lax.axis_index("s")
    wid = cid * NS + sid
    d = rows.shape[1]

    def fill_row(i, _):
        for j in range(d // 16):
            rows[i, pl.ds(j * 16, 16)] = jnp.zeros((16,), F32)
        return 0
    lax.fori_loop(0, CHUNK, fill_row, 0)

    n_per = n_acc // NS
    r_per = r_acc // NS
    for k in range(n_per // CHUNK):
        base = sid * n_per + k * CHUNK
        pltpu.sync_copy(rows, acc_s.at[pl.ds(base, CHUNK)])
    pltpu.sync_copy(rows.at[pl.ds(0, r_per)], acc_r.at[pl.ds(sid * r_per, r_per)])
    plsc.subcore_barrier()

    def chunks(si, di, ei):
        bufs = (rows, rows2, rows3)
        gsems = (sem, sem2, sem3)
        ssems = (ssem, ssem2, ssem3)
        gd = {}
        sd = {}
        gd[0] = pltpu.async_copy(h_hbm.at[si.at[0]], bufs[0], gsems[0])
        for c in range(GROUP):
            b = c % 3
            if c + 1 < GROUP:
                bn = (c + 1) % 3
                if c - 2 >= 0:
                    sd[c - 2][0].wait()
                    sd[c - 2][1].wait()
                gd[c + 1] = pltpu.async_copy(
                    h_hbm.at[si.at[c + 1]], bufs[bn], gsems[bn])
            gd[c].wait()
            sd[c] = (
                pltpu.async_copy(bufs[b], acc_s.at[di.at[c]], ssems[b], add=True),
                pltpu.async_copy(bufs[b], acc_r.at[ei.at[c]], ssems[b], add=True),
            )
        for c in (GROUP - 3, GROUP - 2, GROUP - 1):
            sd[c][0].wait()
            sd[c][1].wait()

    ch0 = (ch * 2 * 3 // 4) // 16 * 16
    ch1 = ch * 2 - ch0

    def run(nch, tile_base):
        def pair(p, _):
            base0 = pl.multiple_of(tile_base + (2 * p) * GROUP, GROUP)
            base1 = pl.multiple_of(tile_base + (2 * p + 1) * GROUP, GROUP)
            pltpu.sync_copy(srci_hbm.at[pl.ds(base0, GROUP)], srci)
            pltpu.sync_copy(dsti_hbm.at[pl.ds(base0, GROUP)], dsti)
            pltpu.sync_copy(eti_hbm.at[pl.ds(base0, GROUP)], eti)
            pf = (
                pltpu.async_copy(srci_hbm.at[pl.ds(base1, GROUP)], srci_b, isem),
                pltpu.async_copy(dsti_hbm.at[pl.ds(base1, GROUP)], dsti_b, isem),
                pltpu.async_copy(eti_hbm.at[pl.ds(base1, GROUP)], eti_b, isem),
            )
            chunks(srci, dsti, eti)
            for x in pf:
                x.wait()
            chunks(srci_b, dsti_b, eti_b)
            return 0
        lax.fori_loop(0, nch // GROUP // 2, pair, 0)

    @pl.when(cid == 0)
    def _():
        run(ch0, sid * ch0)

    @pl.when(cid == 1)
    def _():
        run(ch1, NS * ch0 + sid * ch1)
    plsc.subcore_barrier()

    for k in range(n_per // CHUNK):
        base = sid * n_per + k * CHUNK
        pltpu.sync_copy(acc_s.at[pl.ds(base, CHUNK)], out_s.at[cid, pl.ds(base, CHUNK)])
    pltpu.sync_copy(acc_r.at[pl.ds(sid * r_per, r_per)], out_r.at[cid, pl.ds(sid * r_per, r_per)])


def _sc_pass1(h, srci, dsti, eti, n_acc, r_acc):
    d = h.shape[1]
    ch = srci.shape[0] // NW
    mesh = plsc.VectorSubcoreMesh(core_axis_name="c", subcore_axis_name="s",
                                  num_cores=NC, num_subcores=NS)
    kern = pl.kernel(
        functools.partial(_sc1_body, n_acc, r_acc, ch),
        out_type=(
            jax.ShapeDtypeStruct((NC, n_acc, d), F32),
            jax.ShapeDtypeStruct((NC, r_acc, d), F32),
        ),
        mesh=mesh,
        scratch_types=[
            pltpu.VMEM_SHARED((n_acc, d), F32),
            pltpu.VMEM_SHARED((r_acc, d), F32),
            pltpu.VMEM((GROUP, CHUNK), jnp.int32),
            pltpu.VMEM((GROUP, CHUNK), jnp.int32),
            pltpu.VMEM((GROUP, CHUNK), jnp.int32),
            pltpu.VMEM((GROUP, CHUNK), jnp.int32),
            pltpu.VMEM((GROUP, CHUNK), jnp.int32),
            pltpu.VMEM((GROUP, CHUNK), jnp.int32),
            pltpu.VMEM((CHUNK, d), F32),
            pltpu.VMEM((CHUNK, d), F32),
            pltpu.VMEM((CHUNK, d), F32),
            pltpu.SemaphoreType.DMA,
            pltpu.SemaphoreType.DMA,
            pltpu.SemaphoreType.DMA,
            pltpu.SemaphoreType.DMA,
            pltpu.SemaphoreType.DMA,
            pltpu.SemaphoreType.DMA,
            pltpu.SemaphoreType.DMA,
        ],
    )
    return kern(h, srci, dsti, eti)


CGROUP = 512


def _scc_body(n_acc, hsize, epw,
              dsti_hbm, eti_hbm, out_h,
              idx_d, idx_e, hist):
    cid = lax.axis_index("c")
    sid = lax.axis_index("s")
    wid = cid * NS + sid

    def fill(i, _):
        hist[pl.ds(i * 16, 16)] = jnp.zeros((16,), F32)
        return 0
    lax.fori_loop(0, hsize // 16, fill, 0)

    def group(g, _):
        base = pl.multiple_of(wid * epw + g * CGROUP, 8)
        pltpu.sync_copy(dsti_hbm.at[pl.ds(base, CGROUP)], idx_d)
        pltpu.sync_copy(eti_hbm.at[pl.ds(base, CGROUP)], idx_e)

        def step(k, _):
            dv = idx_d[pl.ds(k * 16, 16)]
            cnts, last = plsc.scan_count(dv)
            plsc.addupdate_scatter(hist, [dv], cnts.astype(F32), mask=last)
            ev = idx_e[pl.ds(k * 16, 16)] + n_acc
            cnts2, last2 = plsc.scan_count(ev)
            plsc.addupdate_scatter(hist, [ev], cnts2.astype(F32), mask=last2)
            return 0
        lax.fori_loop(0, CGROUP // 16, step, 0)
        return 0
    lax.fori_loop(0, epw // CGROUP, group, 0)

    pltpu.sync_copy(hist, out_h.at[pl.ds(wid * hsize, hsize)])


def _sc_counts(dsti_flat, eti_flat, n_acc, r_acc):
    ep = dsti_flat.shape[0]
    epw = ep // NW
    hsize = n_acc + r_acc
    assert epw % CGROUP == 0 and hsize % 16 == 0 and (hsize % 8 == 0)
    mesh = plsc.VectorSubcoreMesh(core_axis_name="c", subcore_axis_name="s",
                                  num_cores=NC, num_subcores=NS)
    kern = pl.kernel(
        functools.partial(_scc_body, n_acc, hsize, epw),
        out_type=jax.ShapeDtypeStruct((NW * hsize,), F32),
        mesh=mesh,
        scratch_types=[
            pltpu.VMEM((CGROUP,), jnp.int32),
            pltpu.VMEM((CGROUP,), jnp.int32),
            pltpu.VMEM((hsize,), F32),
        ],
        compiler_params=pltpu.CompilerParams(needs_layout_passes=False),
    )
    return kern(dsti_flat, eti_flat)


def _sc2_body(n_acc, ch,
              h0_hbm, dsti_hbm, eti_hbm, out_t,
              acc_t, h0_sp, dsti, eti, dsti_b, eti_b,
              rows, rows2, rows3, sem, sem2, sem3,
              ssem, ssem2, ssem3, isem):
    cid = lax.axis_index("c")
    sid = lax.axis_index("s")
    wid = cid * NS + sid
    d = rows.shape[1]

    def fill_row(i, _):
        for j in range(d // 16):
            rows[i, pl.ds(j * 16, 16)] = jnp.zeros((16,), F32)
        return 0
    lax.fori_loop(0, CHUNK, fill_row, 0)

    n_per = n_acc // NS
    for k in range(n_per // CHUNK):
        base = sid * n_per + k * CHUNK
        pltpu.sync_copy(rows, acc_t.at[pl.ds(base, CHUNK)])
    @pl.when(sid == 0)
    def _():
        pltpu.sync_copy(h0_hbm, h0_sp)
    plsc.subcore_barrier()

    def chunks(di, ei):
        bufs = (rows, rows2, rows3)
        gsems = (sem, sem2, sem3)
        ssems = (ssem, ssem2, ssem3)
        gd = {}
        sd = {}
        gd[0] = pltpu.async_copy(h0_sp.at[ei.at[0]], bufs[0], gsems[0])
        for c in range(GROUP):
            b = c % 3
            if c + 1 < GROUP:
                bn = (c + 1) % 3
                if c - 2 >= 0:
                    sd[c - 2].wait()
                gd[c + 1] = pltpu.async_copy(
                    h0_sp.at[ei.at[c + 1]], bufs[bn], gsems[bn])
            gd[c].wait()
            sd[c] = pltpu.async_copy(bufs[b], acc_t.at[di.at[c]], ssems[b], add=True)
        for c in (GROUP - 3, GROUP - 2, GROUP - 1):
            sd[c].wait()

    def pair(p, _):
        base0 = pl.multiple_of(wid * ch + (2 * p) * GROUP, GROUP)
        base1 = pl.multiple_of(wid * ch + (2 * p + 1) * GROUP, GROUP)
        pltpu.sync_copy(dsti_hbm.at[pl.ds(base0, GROUP)], dsti)
        pltpu.sync_copy(eti_hbm.at[pl.ds(base0, GROUP)], eti)
        pf = (
            pltpu.async_copy(dsti_hbm.at[pl.ds(base1, GROUP)], dsti_b, isem),
            pltpu.async_copy(eti_hbm.at[pl.ds(base1, GROUP)], eti_b, isem),
        )
        chunks(dsti, eti)
        for x in pf:
            x.wait()
        chunks(dsti_b, eti_b)
        return 0
    lax.fori_loop(0, ch // GROUP // 2, pair, 0)
    plsc.subcore_barrier()

    for k in range(n_per // CHUNK):
        base = sid * n_per + k * CHUNK
        pltpu.sync_copy(acc_t.at[pl.ds(base, CHUNK)], out_t.at[cid, pl.ds(base, CHUNK)])


def _sc_pass2(h0, dsti, eti, n_acc):
    r_acc, d = h0.shape
    ch = dsti.shape[0] // NW
    mesh = plsc.VectorSubcoreMesh(core_axis_name="c", subcore_axis_name="s",
                                  num_cores=NC, num_subcores=NS)
    kern = pl.kernel(
        functools.partial(_sc2_body, n_acc, ch),
        out_type=jax.ShapeDtypeStruct((NC, n_acc, d), F32),
        mesh=mesh,
        scratch_types=[
            pltpu.VMEM_SHARED((n_acc, d), F32),
            pltpu.VMEM_SHARED((r_acc, d), F32),
            pltpu.VMEM((GROUP, CHUNK), jnp.int32),
            pltpu.VMEM((GROUP, CHUNK), jnp.int32),
            pltpu.VMEM((GROUP, CHUNK), jnp.int32),
            pltpu.VMEM((GROUP, CHUNK), jnp.int32),
            pltpu.VMEM((CHUNK, d), F32),
            pltpu.VMEM((CHUNK, d), F32),
            pltpu.VMEM((CHUNK, d), F32),
            pltpu.SemaphoreType.DMA,
            pltpu.SemaphoreType.DMA,
            pltpu.SemaphoreType.DMA,
            pltpu.SemaphoreType.DMA,
            pltpu.SemaphoreType.DMA,
            pltpu.SemaphoreType.DMA,
            pltpu.SemaphoreType.DMA,
        ],
    )
    return kern(h0, dsti, eti)


def _tc_gru_body(relp_ref, cntp_ref, emb_ref, wih_ref, whh_ref, bih_ref, bhh_ref, o_ref):
    d = emb_ref.shape[1]
    rel_sum = relp_ref[0] + relp_ref[1]
    cnt = jnp.sum(cntp_ref[...], axis=0)
    rel_mean = rel_sum / jnp.maximum(cnt, 1.0)
    emb = emb_ref[...]
    x = jnp.concatenate([emb, rel_mean], axis=1)
    gi = lax.dot_general(x, wih_ref[...], (((1,), (1,)), ((), ())),
                         preferred_element_type=F32) + bih_ref[...]
    gh = lax.dot_general(emb, whh_ref[...], (((1,), (1,)), ((), ())),
                         preferred_element_type=F32) + bhh_ref[...]
    i_r, i_z, i_n = gi[:, :d], gi[:, d:2 * d], gi[:, 2 * d:]
    h_r, h_z, h_n = gh[:, :d], gh[:, d:2 * d], gh[:, 2 * d:]
    r = jax.nn.sigmoid(i_r + h_r)
    z = jax.nn.sigmoid(i_z + h_z)
    n = jnp.tanh(i_n + r * h_n)
    h0 = (1.0 - z) * n + z * emb
    o_ref[...] = _l2norm(h0)


def _tc_gru(rel_p, cnt_p, emb_p, w_ih, w_hh, b_ih, b_hh):
    r_acc, d = emb_p.shape
    return pl.pallas_call(
        _tc_gru_body,
        out_shape=jax.ShapeDtypeStruct((r_acc, d), F32),
    )(rel_p, cnt_p, emb_p, w_ih, w_hh, b_ih, b_hh)


def _tc_final_body(h_ref, s_ref, t_ref, d_ref, wn_ref, lw_ref, tgw_ref, tgb_ref, o_ref):
    u = (s_ref[0] - t_ref[0]) + (s_ref[1] - t_ref[1])
    deg = jnp.sum(d_ref[...], axis=0)
    agg = jnp.dot(u, wn_ref[...], preferred_element_type=F32) / jnp.maximum(deg, 1.0)
    h = h_ref[...]
    cur = agg + jnp.dot(h, lw_ref[...], preferred_element_type=F32)
    slope = (1.0 / 8.0 + 1.0 / 3.0) / 2.0
    cur = jnp.where(cur >= 0, cur, slope * cur)
    cur = _l2norm(cur)
    gate = jax.nn.sigmoid(jnp.dot(cur, tgw_ref[...], preferred_element_type=F32)
                          + tgb_ref[...])
    o_ref[...] = gate * cur + (1.0 - gate) * h


def _tc_final(h, s_p, t_p, d_p, w_n, l_w, tg_w, tg_b, block=1024):
    n_acc, d = h.shape
    grid = n_acc // block
    return pl.pallas_call(
        _tc_final_body,
        grid=(grid,),
        in_specs=[
            pl.BlockSpec((block, d), lambda i: (i, 0)),
            pl.BlockSpec((NC, block, d), lambda i: (0, i, 0)),
            pl.BlockSpec((NC, block, d), lambda i: (0, i, 0)),
            pl.BlockSpec((NW, block, 1), lambda i: (0, i, 0)),
            pl.BlockSpec((d, d), lambda i: (0, 0)),
            pl.BlockSpec((d, d), lambda i: (0, 0)),
            pl.BlockSpec((d, d), lambda i: (0, 0)),
            pl.BlockSpec((1, d), lambda i: (0, 0)),
        ],
        out_specs=pl.BlockSpec((block, d), lambda i: (i, 0)),
        out_shape=jax.ShapeDtypeStruct((n_acc, d), F32),
    )(h, s_p, t_p, d_p, w_n, l_w, tg_w, tg_b)


def kernel(edge_index, edge_type, dynamic_emb, emb_rel, W_ih, W_hh, b_ih, b_hh,
           w_neighbor, loop_weight, time_gate_weight, time_gate_bias):
    n, d = dynamic_emb.shape
    r = emb_rel.shape[0]
    e = edge_type.shape[0]

    ep = ((e + NW * CHUNK * 16 - 1) // (NW * CHUNK * 16)) * (NW * CHUNK * 16)
    pad = ep - e
    n_acc = ((n + NS * CHUNK) // (NS * CHUNK)) * (NS * CHUNK)
    r_acc = ((r + NS - 1) // NS + 1) * NS
    r_acc = ((r_acc + NS * 8 - 1) // (NS * 8)) * (NS * 8)

    src = edge_index[0]
    dst = edge_index[1]
    srci = jnp.concatenate([src, jnp.zeros((pad,), jnp.int32)]).reshape(-1, CHUNK)
    dsti = jnp.concatenate([dst, jnp.full((pad,), n, jnp.int32)]).reshape(-1, CHUNK)
    eti = jnp.concatenate([edge_type, jnp.full((pad,), r, jnp.int32)]).reshape(-1, CHUNK)

    de_p = jnp.zeros((n_acc, d), F32).at[:n].set(dynamic_emb)
    h_pad = _tc_norm(de_p)

    s_p, rel_p = _sc_pass1(h_pad, srci, dsti, eti, n_acc, r_acc)
    hist_flat = _sc_counts(dsti.reshape(-1), eti.reshape(-1), n_acc, r_acc)
    hist_flat = hist_flat.reshape(NW, n_acc + r_acc)
    deg_p = hist_flat[:, :n_acc].reshape(NW, n_acc, 1)
    cnt_p = hist_flat[:, n_acc:].reshape(NW, r_acc, 1)

    emb_p = jnp.zeros((r_acc, d), F32).at[:r].set(emb_rel)
    h0 = _tc_gru(rel_p, cnt_p, emb_p, W_ih, W_hh,
                 b_ih.reshape(1, -1), b_hh.reshape(1, -1))

    t_p = _sc_pass2(h0, dsti, eti, n_acc)

    h_new = _tc_final(h_pad, s_p, t_p, deg_p, w_neighbor, loop_weight,
                      time_gate_weight, time_gate_bias.reshape(1, -1))
    return h_new[:n]
